# Initial kernel scaffold; baseline (speedup 1.0000x reference)
import jax
import jax.numpy as jnp
from jax.experimental import pallas as pl


def kernel(x, edge_index, Ws0, Wn0, b0, Ws1, Wn1, b1, W_mu, b_mu, W_var, b_var, Wd1, bd1, Wd2, bd2, W_scale, b_scale, W_r, b_r, W_do, b_do):
    raise NotImplementedError("write your pallas kernel here")


# speedup vs baseline: 2.9890x; 2.9890x over previous
import functools

import jax
import jax.numpy as jnp
from jax import lax
from jax.experimental import pallas as pl
from jax.experimental.pallas import tpu as pltpu
from jax.experimental.pallas import tpu_sc as plsc

N = 10000
NPAD = 10240
E = 160000
FH = 128
NSUB = 16
NCORE = 2
K = 80
EPT = E // NSUB
NCHUNK = EPT // K
NPR = NPAD // NSUB


@functools.cache
def _get_sc_aggregate():
    mesh = plsc.VectorSubcoreMesh(core_axis_name="c", subcore_axis_name="s",
                                  num_cores=NCORE, num_subcores=NSUB)

    @functools.partial(
        pl.kernel,
        out_type=[
            jax.ShapeDtypeStruct((NCORE, NPAD, FH), jnp.float32),
            jax.ShapeDtypeStruct((NPAD,), jnp.float32),
        ],
        mesh=mesh,
        scratch_types=[
            pltpu.VMEM((K,), jnp.int32),
            pltpu.VMEM((K,), jnp.int32),
            pltpu.VMEM((K,), jnp.int32),
            pltpu.VMEM((K, FH), jnp.float32),
            pltpu.VMEM((K,), jnp.float32),
            pltpu.VMEM((NPR,), jnp.float32),
            pltpu.VMEM_SHARED((NPAD, FH), jnp.float32),
            pltpu.VMEM_SHARED((NPAD,), jnp.float32),
            pltpu.SemaphoreType.DMA,
        ],
    )
    def _sc_aggregate(h_hbm, src_hbm, dst_hbm, sums_hbm, deg_hbm,
                      src_v, dst_v, srcadj_v, rows_v, ones_v, zdeg_v,
                      acc_sh, deg_sh, gsem):
        c = lax.axis_index("c")
        s = lax.axis_index("s")

        def _zrow(i, carry):
            for j in range(FH // 16):
                rows_v[i, pl.ds(j * 16, 16)] = jnp.zeros((16,), jnp.float32)
            return carry
        lax.fori_loop(0, K, _zrow, 0)
        for j in range(K // 16):
            ones_v[pl.ds(j * 16, 16)] = jnp.ones((16,), jnp.float32)

        def _zdeg(i, carry):
            zdeg_v[pl.ds(i * 16, 16)] = jnp.zeros((16,), jnp.float32)
            return carry
        lax.fori_loop(0, NPR // 16, _zdeg, 0)

        for r in range(NPR // K):
            pltpu.sync_copy(rows_v, acc_sh.at[pl.ds(s * NPR + r * K, K), :])
        pltpu.sync_copy(zdeg_v, deg_sh.at[pl.ds(s * NPR, NPR)])
        plsc.subcore_barrier()

        ebase = s * EPT
        coff = c * NPAD

        def _chunk(i, carry):
            off = ebase + i * K
            pltpu.sync_copy(src_hbm.at[pl.ds(off, K)], src_v)
            pltpu.sync_copy(dst_hbm.at[pl.ds(off, K)], dst_v)
            for j in range(K // 16):
                srcadj_v[pl.ds(j * 16, 16)] = src_v[pl.ds(j * 16, 16)] + coff
            pltpu.async_copy(h_hbm.at[srcadj_v], rows_v, gsem).wait()
            pltpu.sync_copy(rows_v, acc_sh.at[dst_v], add=True)
            pltpu.sync_copy(ones_v, deg_sh.at[dst_v], add=True)
            return carry
        lax.fori_loop(0, NCHUNK, _chunk, 0)
        plsc.subcore_barrier()

        def _wb(r, carry):
            base = s * NPR + r * K
            pltpu.sync_copy(acc_sh.at[pl.ds(base, K), :], rows_v)
            pltpu.sync_copy(rows_v, sums_hbm.at[c, pl.ds(base, K), :])
            return carry
        lax.fori_loop(0, NPR // K, _wb, 0)

        @pl.when(c == 0)
        def _():
            pltpu.sync_copy(deg_sh.at[pl.ds(s * NPR, NPR)], zdeg_v)
            pltpu.sync_copy(zdeg_v, deg_hbm.at[pl.ds(s * NPR, NPR)])

    return _sc_aggregate


_BM = 512


def _pre_body(x_ref, out_ref):
    h = jnp.log(x_ref[...] + 1.0)
    out_ref[0] = h[:, :FH]
    out_ref[1] = h[:, FH:]


def _tc_pre(x_pad):
    return pl.pallas_call(
        _pre_body,
        grid=(NPAD // _BM,),
        in_specs=[pl.BlockSpec((_BM, 2 * FH), lambda i: (i, 0))],
        out_specs=pl.BlockSpec((2, _BM, FH), lambda i: (0, i, 0)),
        out_shape=jax.ShapeDtypeStruct((2, NPAD, FH), jnp.float32),
    )(x_pad)


def _dot(a, b):
    return jnp.dot(a, b, preferred_element_type=jnp.float32)


def _layer_math(h_ref, s_ref, deg_ref, Ws_ref, Wn_ref, b_ref):
    inv = 1.0 / jnp.maximum(deg_ref[...], 1.0)
    out = (_dot(h_ref[0], Ws_ref[0]) + _dot(h_ref[1], Ws_ref[1])
           + _dot(s_ref[0] * inv, Wn_ref[0]) + _dot(s_ref[1] * inv, Wn_ref[1])
           + b_ref[...])
    out = jnp.maximum(out, 0.0)
    nrm = jnp.sqrt(jnp.sum(out * out, axis=1, keepdims=True))
    return out / jnp.maximum(nrm, 1e-12)


def _layer_body(h_ref, s_ref, deg_ref, Ws_ref, Wn_ref, b_ref, out_ref):
    out = _layer_math(h_ref, s_ref, deg_ref, Ws_ref, Wn_ref, b_ref)
    out_ref[0] = out[:, :FH]
    out_ref[1] = out[:, FH:]


def _tc_layer(h, sums, deg2d, Ws, Wn, b):
    w_spec = pl.BlockSpec((2, FH, 2 * FH), lambda i: (0, 0, 0))
    return pl.pallas_call(
        _layer_body,
        grid=(NPAD // _BM,),
        in_specs=[
            pl.BlockSpec((2, _BM, FH), lambda i: (0, i, 0)),
            pl.BlockSpec((2, _BM, FH), lambda i: (0, i, 0)),
            pl.BlockSpec((_BM, 1), lambda i: (i, 0)),
            w_spec, w_spec,
            pl.BlockSpec((1, 2 * FH), lambda i: (0, 0)),
        ],
        out_specs=pl.BlockSpec((2, _BM, FH), lambda i: (0, i, 0)),
        out_shape=jax.ShapeDtypeStruct((2, NPAD, FH), jnp.float32),
    )(h, sums, deg2d, Ws, Wn, b)


def _head_body(h_ref,
               Wmu_ref, bmu_ref, Wvar_ref, bvar_ref,
               Wd1_ref, bd1_ref, Wd2_ref, bd2_ref,
               Wsc_ref, bsc_ref, Wr_ref, br_ref, Wdo_ref, bdo_ref,
               zloc_ref, zscale_ref, pxs_ref, pxr_ref, pxl_ref):
    h2 = jnp.concatenate([h_ref[0], h_ref[1]], axis=1)
    z_loc = _dot(h2, Wmu_ref[...]) + bmu_ref[...]
    zloc_ref[...] = z_loc
    zscale_ref[...] = jnp.exp(_dot(h2, Wvar_ref[...]) + bvar_ref[...]) + 1e-6
    px = jnp.maximum(_dot(z_loc, Wd1_ref[...]) + bd1_ref[...], 0.0)
    px = jnp.maximum(_dot(px, Wd2_ref[...]) + bd2_ref[...], 0.0)
    t = _dot(px, Wsc_ref[...]) + bsc_ref[...]
    t = t - jnp.max(t, axis=1, keepdims=True)
    e = jnp.exp(t)
    pxs_ref[...] = e / jnp.sum(e, axis=1, keepdims=True)
    pxr_ref[...] = _dot(px, Wr_ref[...]) + br_ref[...]
    pxl_ref[...] = _dot(px, Wdo_ref[...]) + bdo_ref[...]


def _full_spec(shape):
    nd = len(shape)
    return pl.BlockSpec(shape, lambda i, _nd=nd: (0,) * _nd)


def _tc_head(h, W_mu, b_mu, W_var, b_var,
             Wd1, bd1, Wd2, bd2, W_scale, b_scale, W_r, b_r, W_do, b_do):
    L = W_mu.shape[1]
    C = W_scale.shape[1]
    F = W_r.shape[1]
    return pl.pallas_call(
        _head_body,
        grid=(NPAD // _BM,),
        in_specs=[
            pl.BlockSpec((2, _BM, FH), lambda i: (0, i, 0)),
            _full_spec(W_mu.shape), _full_spec((1, L)),
            _full_spec(W_var.shape), _full_spec((1, L)),
            _full_spec(Wd1.shape), _full_spec((1, 2 * FH)),
            _full_spec(Wd2.shape), _full_spec((1, 2 * FH)),
            _full_spec(W_scale.shape), _full_spec((1, C)),
            _full_spec(W_r.shape), _full_spec((1, F)),
            _full_spec(W_do.shape), _full_spec((1, 1)),
        ],
        out_specs=[
            pl.BlockSpec((_BM, L), lambda i: (i, 0)),
            pl.BlockSpec((_BM, L), lambda i: (i, 0)),
            pl.BlockSpec((_BM, C), lambda i: (i, 0)),
            pl.BlockSpec((_BM, F), lambda i: (i, 0)),
            pl.BlockSpec((_BM, 1), lambda i: (i, 0)),
        ],
        out_shape=[
            jax.ShapeDtypeStruct((NPAD, L), jnp.float32),
            jax.ShapeDtypeStruct((NPAD, L), jnp.float32),
            jax.ShapeDtypeStruct((NPAD, C), jnp.float32),
            jax.ShapeDtypeStruct((NPAD, F), jnp.float32),
            jax.ShapeDtypeStruct((NPAD, 1), jnp.float32),
        ],
    )(h, W_mu, b_mu.reshape(1, L),
      W_var, b_var.reshape(1, L), Wd1, bd1.reshape(1, -1),
      Wd2, bd2.reshape(1, -1), W_scale, b_scale.reshape(1, C),
      W_r, b_r.reshape(1, F), W_do, b_do.reshape(1, 1))


def kernel(x, edge_index, Ws0, Wn0, b0, Ws1, Wn1, b1, W_mu, b_mu, W_var,
           b_var, Wd1, bd1, Wd2, bd2, W_scale, b_scale, W_r, b_r, W_do, b_do):
    src = edge_index[0]
    dst = edge_index[1]
    x_pad = jnp.pad(x, ((0, NPAD - N), (0, 0)))
    sc_aggregate = _get_sc_aggregate()

    h0 = _tc_pre(x_pad)

    Wss = jnp.stack([Ws0.reshape(2, FH, 2 * FH), Ws1.reshape(2, FH, 2 * FH)])
    Wns = jnp.stack([Wn0.reshape(2, FH, 2 * FH), Wn1.reshape(2, FH, 2 * FH)])
    bs = jnp.stack([b0.reshape(1, 2 * FH), b1.reshape(1, 2 * FH)])

    def _scan_body(h, wk):
        Ws, Wn, b = wk
        sums, deg = sc_aggregate(h.reshape(2 * NPAD, FH), src, dst)
        h_new = _tc_layer(h, sums, deg.reshape(NPAD, 1), Ws, Wn, b)
        return h_new, None

    h2, _ = lax.scan(_scan_body, h0, (Wss, Wns, bs))
    z_loc, z_scale, px_scale, px_r, px_l = _tc_head(
        h2, W_mu, b_mu, W_var, b_var, Wd1, bd1, Wd2, bd2,
        W_scale, b_scale, W_r, b_r, W_do, b_do)
    return (z_loc[:N], z_scale[:N], px_scale[:N], px_r[:N], px_l[:N])


# speedup vs baseline: 5.6492x; 1.8900x over previous
import functools

import jax
import jax.numpy as jnp
from jax import lax
from jax.experimental import pallas as pl
from jax.experimental.pallas import tpu as pltpu
from jax.experimental.pallas import tpu_sc as plsc

N = 10000
NPAD = 10240
E = 160000
FH = 128
NSUB = 16
NCORE = 2
K = 80
EPT = E // NSUB
NCHUNK = EPT // K
NPR = NPAD // NSUB


@functools.cache
def _get_sc_aggregate():
    mesh = plsc.VectorSubcoreMesh(core_axis_name="c", subcore_axis_name="s",
                                  num_cores=NCORE, num_subcores=NSUB)

    @functools.partial(
        pl.kernel,
        out_type=[
            jax.ShapeDtypeStruct((NCORE, NPAD, FH), jnp.float32),
            jax.ShapeDtypeStruct((NPAD,), jnp.float32),
        ],
        mesh=mesh,
        scratch_types=[
            pltpu.VMEM((NCHUNK, K), jnp.int32),
            pltpu.VMEM((2, K), jnp.int32),
            pltpu.VMEM((2, K, FH), jnp.float32),
            pltpu.VMEM((K,), jnp.float32),
            pltpu.VMEM((NPR,), jnp.float32),
            pltpu.VMEM_SHARED((NPAD, FH), jnp.float32),
            pltpu.VMEM_SHARED((NPAD,), jnp.float32),
            pltpu.SemaphoreType.DMA,
            pltpu.SemaphoreType.DMA,
            pltpu.SemaphoreType.DMA,
            pltpu.SemaphoreType.DMA,
            pltpu.SemaphoreType.DMA,
        ],
    )
    def _sc_aggregate(h_hbm, src_hbm, dst_hbm, sums_hbm, deg_hbm,
                      dst_all, srcadj_v, rows_v, ones_v, zdeg_v,
                      acc_sh, deg_sh, gsem0, gsem1, ssem, dsem, isem):
        c = lax.axis_index("c")
        s = lax.axis_index("s")
        gsems = (gsem0, gsem1)

        pltpu.sync_copy(dst_hbm.at[s], dst_all)

        def _zrow(i, carry):
            for j in range(FH // 16):
                rows_v[0, i, pl.ds(j * 16, 16)] = jnp.zeros((16,), jnp.float32)
            return carry
        lax.fori_loop(0, K, _zrow, 0)
        for j in range(K // 16):
            ones_v[pl.ds(j * 16, 16)] = jnp.ones((16,), jnp.float32)

        def _zdeg(i, carry):
            zdeg_v[pl.ds(i * 16, 16)] = jnp.zeros((16,), jnp.float32)
            return carry
        lax.fori_loop(0, NPR // 16, _zdeg, 0)

        for r in range(NPR // K):
            pltpu.sync_copy(rows_v.at[0],
                            acc_sh.at[pl.ds(s * NPR + r * K, K), :])
        pltpu.sync_copy(zdeg_v, deg_sh.at[pl.ds(s * NPR, NPR)])
        plsc.subcore_barrier()

        coff = c * NPAD

        def _adjust(b):
            for j in range(K // 16):
                srcadj_v[b, pl.ds(j * 16, 16)] = (
                    srcadj_v[b, pl.ds(j * 16, 16)] + coff)

        def _gather_start(b):
            pltpu.async_copy(h_hbm.at[srcadj_v.at[b]], rows_v.at[b], gsems[b])

        def _gather_wait(b):
            pltpu.make_async_copy(h_hbm.at[srcadj_v.at[b]], rows_v.at[b],
                                  gsems[b]).wait()

        for b in (0, 1):
            pltpu.sync_copy(src_hbm.at[s, b], srcadj_v.at[b])
            _adjust(b)
            _gather_start(b)

        def _consume(i, b, nxt):
            _gather_wait(b)
            if nxt:
                ix = pltpu.async_copy(src_hbm.at[s, i + 2], srcadj_v.at[b],
                                      isem)
            sc = pltpu.async_copy(rows_v.at[b], acc_sh.at[dst_all.at[i]],
                                  ssem, add=True)
            dg = pltpu.async_copy(ones_v, deg_sh.at[dst_all.at[i]],
                                  dsem, add=True)
            if nxt:
                ix.wait()
                _adjust(b)
            sc.wait()
            dg.wait()
            if nxt:
                _gather_start(b)

        def _pair(p, carry):
            _consume(2 * p, 0, True)

            @pl.when(p < (NCHUNK - 3) // 2)
            def _():
                _consume(2 * p + 1, 1, True)

            @pl.when(p >= (NCHUNK - 3) // 2)
            def _():
                _consume(2 * p + 1, 1, False)
            return carry
        lax.fori_loop(0, (NCHUNK - 1) // 2, _pair, 0)
        _consume(NCHUNK - 1, 0, False)
        plsc.subcore_barrier()

        def _wb(r, carry):
            base = s * NPR + r * K
            pltpu.sync_copy(acc_sh.at[pl.ds(base, K), :], rows_v.at[0])
            pltpu.sync_copy(rows_v.at[0], sums_hbm.at[c, pl.ds(base, K), :])
            return carry
        lax.fori_loop(0, NPR // K, _wb, 0)

        @pl.when(c == 0)
        def _():
            pltpu.sync_copy(deg_sh.at[pl.ds(s * NPR, NPR)], zdeg_v)
            pltpu.sync_copy(zdeg_v, deg_hbm.at[pl.ds(s * NPR, NPR)])

    return _sc_aggregate


_BM = 512


def _pre_body(x_ref, out_ref):
    h = jnp.log(x_ref[...] + 1.0)
    out_ref[0] = h[:, :FH]
    out_ref[1] = h[:, FH:]


def _tc_pre(x_pad):
    return pl.pallas_call(
        _pre_body,
        grid=(NPAD // _BM,),
        in_specs=[pl.BlockSpec((_BM, 2 * FH), lambda i: (i, 0))],
        out_specs=pl.BlockSpec((2, _BM, FH), lambda i: (0, i, 0)),
        out_shape=jax.ShapeDtypeStruct((2, NPAD, FH), jnp.float32),
    )(x_pad)


def _dot(a, b):
    return jnp.dot(a, b, preferred_element_type=jnp.float32)


def _layer_math(h_ref, s_ref, deg_ref, Ws_ref, Wn_ref, b_ref):
    inv = 1.0 / jnp.maximum(deg_ref[...], 1.0)
    out = (_dot(h_ref[0], Ws_ref[0]) + _dot(h_ref[1], Ws_ref[1])
           + _dot(s_ref[0] * inv, Wn_ref[0]) + _dot(s_ref[1] * inv, Wn_ref[1])
           + b_ref[...])
    out = jnp.maximum(out, 0.0)
    nrm = jnp.sqrt(jnp.sum(out * out, axis=1, keepdims=True))
    return out / jnp.maximum(nrm, 1e-12)


def _layer_body(h_ref, s_ref, deg_ref, Ws_ref, Wn_ref, b_ref, out_ref):
    out = _layer_math(h_ref, s_ref, deg_ref, Ws_ref, Wn_ref, b_ref)
    out_ref[0] = out[:, :FH]
    out_ref[1] = out[:, FH:]


def _tc_layer(h, sums, deg2d, Ws, Wn, b):
    w_spec = pl.BlockSpec((2, FH, 2 * FH), lambda i: (0, 0, 0))
    return pl.pallas_call(
        _layer_body,
        grid=(NPAD // _BM,),
        in_specs=[
            pl.BlockSpec((2, _BM, FH), lambda i: (0, i, 0)),
            pl.BlockSpec((2, _BM, FH), lambda i: (0, i, 0)),
            pl.BlockSpec((_BM, 1), lambda i: (i, 0)),
            w_spec, w_spec,
            pl.BlockSpec((1, 2 * FH), lambda i: (0, 0)),
        ],
        out_specs=pl.BlockSpec((2, _BM, FH), lambda i: (0, i, 0)),
        out_shape=jax.ShapeDtypeStruct((2, NPAD, FH), jnp.float32),
    )(h, sums, deg2d, Ws, Wn, b)


def _head_body(h_ref,
               Wmu_ref, bmu_ref, Wvar_ref, bvar_ref,
               Wd1_ref, bd1_ref, Wd2_ref, bd2_ref,
               Wsc_ref, bsc_ref, Wr_ref, br_ref, Wdo_ref, bdo_ref,
               zloc_ref, zscale_ref, pxs_ref, pxr_ref, pxl_ref):
    h2 = jnp.concatenate([h_ref[0], h_ref[1]], axis=1)
    z_loc = _dot(h2, Wmu_ref[...]) + bmu_ref[...]
    zloc_ref[...] = z_loc
    zscale_ref[...] = jnp.exp(_dot(h2, Wvar_ref[...]) + bvar_ref[...]) + 1e-6
    px = jnp.maximum(_dot(z_loc, Wd1_ref[...]) + bd1_ref[...], 0.0)
    px = jnp.maximum(_dot(px, Wd2_ref[...]) + bd2_ref[...], 0.0)
    t = _dot(px, Wsc_ref[...]) + bsc_ref[...]
    t = t - jnp.max(t, axis=1, keepdims=True)
    e = jnp.exp(t)
    pxs_ref[...] = e / jnp.sum(e, axis=1, keepdims=True)
    pxr_ref[...] = _dot(px, Wr_ref[...]) + br_ref[...]
    pxl_ref[...] = _dot(px, Wdo_ref[...]) + bdo_ref[...]


def _full_spec(shape):
    nd = len(shape)
    return pl.BlockSpec(shape, lambda i, _nd=nd: (0,) * _nd)


def _tc_head(h, W_mu, b_mu, W_var, b_var,
             Wd1, bd1, Wd2, bd2, W_scale, b_scale, W_r, b_r, W_do, b_do):
    L = W_mu.shape[1]
    C = W_scale.shape[1]
    F = W_r.shape[1]
    return pl.pallas_call(
        _head_body,
        grid=(NPAD // _BM,),
        in_specs=[
            pl.BlockSpec((2, _BM, FH), lambda i: (0, i, 0)),
            _full_spec(W_mu.shape), _full_spec((1, L)),
            _full_spec(W_var.shape), _full_spec((1, L)),
            _full_spec(Wd1.shape), _full_spec((1, 2 * FH)),
            _full_spec(Wd2.shape), _full_spec((1, 2 * FH)),
            _full_spec(W_scale.shape), _full_spec((1, C)),
            _full_spec(W_r.shape), _full_spec((1, F)),
            _full_spec(W_do.shape), _full_spec((1, 1)),
        ],
        out_specs=[
            pl.BlockSpec((_BM, L), lambda i: (i, 0)),
            pl.BlockSpec((_BM, L), lambda i: (i, 0)),
            pl.BlockSpec((_BM, C), lambda i: (i, 0)),
            pl.BlockSpec((_BM, F), lambda i: (i, 0)),
            pl.BlockSpec((_BM, 1), lambda i: (i, 0)),
        ],
        out_shape=[
            jax.ShapeDtypeStruct((NPAD, L), jnp.float32),
            jax.ShapeDtypeStruct((NPAD, L), jnp.float32),
            jax.ShapeDtypeStruct((NPAD, C), jnp.float32),
            jax.ShapeDtypeStruct((NPAD, F), jnp.float32),
            jax.ShapeDtypeStruct((NPAD, 1), jnp.float32),
        ],
    )(h, W_mu, b_mu.reshape(1, L),
      W_var, b_var.reshape(1, L), Wd1, bd1.reshape(1, -1),
      Wd2, bd2.reshape(1, -1), W_scale, b_scale.reshape(1, C),
      W_r, b_r.reshape(1, F), W_do, b_do.reshape(1, 1))


def kernel(x, edge_index, Ws0, Wn0, b0, Ws1, Wn1, b1, W_mu, b_mu, W_var,
           b_var, Wd1, bd1, Wd2, bd2, W_scale, b_scale, W_r, b_r, W_do, b_do):
    src = edge_index[0].reshape(NSUB, NCHUNK, K)
    dst = edge_index[1].reshape(NSUB, NCHUNK, K)
    x_pad = jnp.pad(x, ((0, NPAD - N), (0, 0)))
    sc_aggregate = _get_sc_aggregate()

    h0 = _tc_pre(x_pad)

    Wss = jnp.stack([Ws0.reshape(2, FH, 2 * FH), Ws1.reshape(2, FH, 2 * FH)])
    Wns = jnp.stack([Wn0.reshape(2, FH, 2 * FH), Wn1.reshape(2, FH, 2 * FH)])
    bs = jnp.stack([b0.reshape(1, 2 * FH), b1.reshape(1, 2 * FH)])

    def _scan_body(h, wk):
        Ws, Wn, b = wk
        sums, deg = sc_aggregate(h.reshape(2 * NPAD, FH), src, dst)
        h_new = _tc_layer(h, sums, deg.reshape(NPAD, 1), Ws, Wn, b)
        return h_new, None

    h2, _ = lax.scan(_scan_body, h0, (Wss, Wns, bs))
    z_loc, z_scale, px_scale, px_r, px_l = _tc_head(
        h2, W_mu, b_mu, W_var, b_var, Wd1, bd1, Wd2, bd2,
        W_scale, b_scale, W_r, b_r, W_do, b_do)
    return (z_loc[:N], z_scale[:N], px_scale[:N], px_r[:N], px_l[:N])


# speedup vs baseline: 5.7647x; 1.0205x over previous
import functools

import jax
import jax.numpy as jnp
from jax import lax
from jax.experimental import pallas as pl
from jax.experimental.pallas import tpu as pltpu
from jax.experimental.pallas import tpu_sc as plsc

N = 10000
NPAD = 10240
E = 160000
FH = 128
NSUB = 16
NCORE = 2
K = 80
EPT = E // NSUB
NCHUNK = EPT // K
NPR = NPAD // NSUB


@functools.cache
def _get_sc_aggregate():
    mesh = plsc.VectorSubcoreMesh(core_axis_name="c", subcore_axis_name="s",
                                  num_cores=NCORE, num_subcores=NSUB)

    @functools.partial(
        pl.kernel,
        out_type=[
            jax.ShapeDtypeStruct((NCORE, NPAD, FH), jnp.float32),
            jax.ShapeDtypeStruct((NPAD,), jnp.float32),
        ],
        mesh=mesh,
        scratch_types=[
            pltpu.VMEM((3, K), jnp.int32),
            pltpu.VMEM((3, K), jnp.int32),
            pltpu.VMEM((3, K, FH), jnp.float32),
            pltpu.VMEM((K,), jnp.float32),
            pltpu.VMEM((NPR,), jnp.float32),
            pltpu.VMEM_SHARED((NPAD, FH), jnp.float32),
            pltpu.VMEM_SHARED((NPAD,), jnp.float32),
        ] + [pltpu.SemaphoreType.DMA] * 15,
    )
    def _sc_aggregate(h_hbm, src_hbm, dst_hbm, sums_hbm, deg_hbm,
                      srcadj_v, dstb_v, rows_v, ones_v, zdeg_v,
                      acc_sh, deg_sh, *sems):
        c = lax.axis_index("c")
        s = lax.axis_index("s")
        gsems = sems[0:3]
        ssems = sems[3:6]
        xsems = sems[6:9]
        dxsems = sems[9:12]
        dsems = sems[12:15]

        def _zrow(i, carry):
            for j in range(FH // 16):
                rows_v[0, i, pl.ds(j * 16, 16)] = jnp.zeros((16,), jnp.float32)
            return carry
        lax.fori_loop(0, K, _zrow, 0)
        for j in range(K // 16):
            ones_v[pl.ds(j * 16, 16)] = jnp.ones((16,), jnp.float32)

        def _zdeg(i, carry):
            zdeg_v[pl.ds(i * 16, 16)] = jnp.zeros((16,), jnp.float32)
            return carry
        lax.fori_loop(0, NPR // 16, _zdeg, 0)

        for r in range(NPR // K):
            pltpu.sync_copy(rows_v.at[0],
                            acc_sh.at[pl.ds(s * NPR + r * K, K), :])
        pltpu.sync_copy(zdeg_v, deg_sh.at[pl.ds(s * NPR, NPR)])
        plsc.subcore_barrier()

        coff = c * NPAD

        def _adjust(b):
            for j in range(K // 16):
                srcadj_v[b, pl.ds(j * 16, 16)] = (
                    srcadj_v[b, pl.ds(j * 16, 16)] + coff)

        def _gather_start(b):
            pltpu.async_copy(h_hbm.at[srcadj_v.at[b]], rows_v.at[b], gsems[b])

        def _gather_wait(b):
            pltpu.make_async_copy(h_hbm.at[srcadj_v.at[b]], rows_v.at[b],
                                  gsems[b]).wait()

        def _scatter_start(i, b):
            pltpu.async_copy(rows_v.at[b], acc_sh.at[dstb_v.at[b]],
                             ssems[b], add=True)
            pltpu.async_copy(ones_v, deg_sh.at[dstb_v.at[b]],
                             dsems[b], add=True)

        def _scatter_wait(b):
            pltpu.make_async_copy(rows_v.at[b], acc_sh.at[dstb_v.at[b]],
                                  ssems[b]).wait()
            pltpu.make_async_copy(ones_v, deg_sh.at[dstb_v.at[b]],
                                  dsems[b]).wait()

        def _srcidx_start(i, b):
            pltpu.async_copy(src_hbm.at[pl.ds(s * EPT + i * K, K)],
                             srcadj_v.at[b], xsems[b])

        def _srcidx_wait(i, b):
            pltpu.make_async_copy(src_hbm.at[pl.ds(s * EPT + i * K, K)],
                                  srcadj_v.at[b], xsems[b]).wait()

        def _dstidx_start(i, b):
            pltpu.async_copy(dst_hbm.at[pl.ds(s * EPT + i * K, K)],
                             dstb_v.at[b], dxsems[b])

        def _dstidx_wait(i, b):
            pltpu.make_async_copy(dst_hbm.at[pl.ds(s * EPT + i * K, K)],
                                  dstb_v.at[b], dxsems[b]).wait()

        for b in (0, 1, 2):
            _srcidx_start(b, b)
            _dstidx_start(b, b)
            _srcidx_wait(b, b)
            _adjust(b)
            if b < 2:
                _gather_start(b)

        def _section(i, b, first, do_src, do_next):
            _gather_wait(b)
            _dstidx_wait(i, b)
            _scatter_start(i, b)
            if do_src:
                _srcidx_start(i + 3, b)
                _srcidx_wait(i + 3, b)
                _adjust(b)
            if not first:
                bp = (b + 2) % 3
                _scatter_wait(bp)
                if do_next:
                    _dstidx_start(i + 2, bp)
                    _gather_start(bp)

        _gather_wait(0)
        _dstidx_wait(0, 0)
        _scatter_start(0, 0)
        _srcidx_start(3, 0)
        _srcidx_wait(3, 0)
        _adjust(0)
        _gather_start(2)
        _section(1, 1, False, True, True)
        _section(2, 2, False, True, True)

        def _trip(p, carry):
            _section(3 * p, 0, False, True, True)
            _section(3 * p + 1, 1, False, True, True)

            @pl.when(p < (NCHUNK - 5) // 3)
            def _():
                _section(3 * p + 2, 2, False, True, True)

            @pl.when(p >= (NCHUNK - 5) // 3)
            def _():
                _section(3 * p + 2, 2, False, False, True)
            return carry
        lax.fori_loop(1, (NCHUNK - 2) // 3, _trip, 0)
        _section(NCHUNK - 2, 0, False, False, False)
        _section(NCHUNK - 1, 1, False, False, False)
        _scatter_wait(1)
        plsc.subcore_barrier()

        def _wb(r, carry):
            base = s * NPR + r * K
            pltpu.sync_copy(acc_sh.at[pl.ds(base, K), :], rows_v.at[0])
            pltpu.sync_copy(rows_v.at[0], sums_hbm.at[c, pl.ds(base, K), :])
            return carry
        lax.fori_loop(0, NPR // K, _wb, 0)

        @pl.when(c == 0)
        def _():
            pltpu.sync_copy(deg_sh.at[pl.ds(s * NPR, NPR)], zdeg_v)
            pltpu.sync_copy(zdeg_v, deg_hbm.at[pl.ds(s * NPR, NPR)])

    return _sc_aggregate


_BM = 512


def _pre_body(x_ref, out_ref):
    h = jnp.log(x_ref[...] + 1.0)
    out_ref[0] = h[:, :FH]
    out_ref[1] = h[:, FH:]


def _tc_pre(x_pad):
    return pl.pallas_call(
        _pre_body,
        grid=(NPAD // _BM,),
        in_specs=[pl.BlockSpec((_BM, 2 * FH), lambda i: (i, 0))],
        out_specs=pl.BlockSpec((2, _BM, FH), lambda i: (0, i, 0)),
        out_shape=jax.ShapeDtypeStruct((2, NPAD, FH), jnp.float32),
    )(x_pad)


def _dot(a, b):
    return jnp.dot(a, b, preferred_element_type=jnp.float32)


def _layer_math(h_ref, s_ref, deg_ref, Ws_ref, Wn_ref, b_ref):
    inv = 1.0 / jnp.maximum(deg_ref[...], 1.0)
    out = (_dot(h_ref[0], Ws_ref[0]) + _dot(h_ref[1], Ws_ref[1])
           + _dot(s_ref[0] * inv, Wn_ref[0]) + _dot(s_ref[1] * inv, Wn_ref[1])
           + b_ref[...])
    out = jnp.maximum(out, 0.0)
    nrm = jnp.sqrt(jnp.sum(out * out, axis=1, keepdims=True))
    return out / jnp.maximum(nrm, 1e-12)


def _layer_body(h_ref, s_ref, deg_ref, Ws_ref, Wn_ref, b_ref, out_ref):
    out = _layer_math(h_ref, s_ref, deg_ref, Ws_ref, Wn_ref, b_ref)
    out_ref[0] = out[:, :FH]
    out_ref[1] = out[:, FH:]


def _tc_layer(h, sums, deg2d, Ws, Wn, b):
    w_spec = pl.BlockSpec((2, FH, 2 * FH), lambda i: (0, 0, 0))
    return pl.pallas_call(
        _layer_body,
        grid=(NPAD // _BM,),
        in_specs=[
            pl.BlockSpec((2, _BM, FH), lambda i: (0, i, 0)),
            pl.BlockSpec((2, _BM, FH), lambda i: (0, i, 0)),
            pl.BlockSpec((_BM, 1), lambda i: (i, 0)),
            w_spec, w_spec,
            pl.BlockSpec((1, 2 * FH), lambda i: (0, 0)),
        ],
        out_specs=pl.BlockSpec((2, _BM, FH), lambda i: (0, i, 0)),
        out_shape=jax.ShapeDtypeStruct((2, NPAD, FH), jnp.float32),
    )(h, sums, deg2d, Ws, Wn, b)


def _head_body(h_ref,
               Wmu_ref, bmu_ref, Wvar_ref, bvar_ref,
               Wd1_ref, bd1_ref, Wd2_ref, bd2_ref,
               Wsc_ref, bsc_ref, Wr_ref, br_ref, Wdo_ref, bdo_ref,
               zloc_ref, zscale_ref, pxs_ref, pxr_ref, pxl_ref):
    h2 = jnp.concatenate([h_ref[0], h_ref[1]], axis=1)
    z_loc = _dot(h2, Wmu_ref[...]) + bmu_ref[...]
    zloc_ref[...] = z_loc
    zscale_ref[...] = jnp.exp(_dot(h2, Wvar_ref[...]) + bvar_ref[...]) + 1e-6
    px = jnp.maximum(_dot(z_loc, Wd1_ref[...]) + bd1_ref[...], 0.0)
    px = jnp.maximum(_dot(px, Wd2_ref[...]) + bd2_ref[...], 0.0)
    t = _dot(px, Wsc_ref[...]) + bsc_ref[...]
    t = t - jnp.max(t, axis=1, keepdims=True)
    e = jnp.exp(t)
    pxs_ref[...] = e / jnp.sum(e, axis=1, keepdims=True)
    pxr_ref[...] = _dot(px, Wr_ref[...]) + br_ref[...]
    pxl_ref[...] = _dot(px, Wdo_ref[...]) + bdo_ref[...]


def _full_spec(shape):
    nd = len(shape)
    return pl.BlockSpec(shape, lambda i, _nd=nd: (0,) * _nd)


def _tc_head(h, W_mu, b_mu, W_var, b_var,
             Wd1, bd1, Wd2, bd2, W_scale, b_scale, W_r, b_r, W_do, b_do):
    L = W_mu.shape[1]
    C = W_scale.shape[1]
    F = W_r.shape[1]
    return pl.pallas_call(
        _head_body,
        grid=(NPAD // _BM,),
        in_specs=[
            pl.BlockSpec((2, _BM, FH), lambda i: (0, i, 0)),
            _full_spec(W_mu.shape), _full_spec((1, L)),
            _full_spec(W_var.shape), _full_spec((1, L)),
            _full_spec(Wd1.shape), _full_spec((1, 2 * FH)),
            _full_spec(Wd2.shape), _full_spec((1, 2 * FH)),
            _full_spec(W_scale.shape), _full_spec((1, C)),
            _full_spec(W_r.shape), _full_spec((1, F)),
            _full_spec(W_do.shape), _full_spec((1, 1)),
        ],
        out_specs=[
            pl.BlockSpec((_BM, L), lambda i: (i, 0)),
            pl.BlockSpec((_BM, L), lambda i: (i, 0)),
            pl.BlockSpec((_BM, C), lambda i: (i, 0)),
            pl.BlockSpec((_BM, F), lambda i: (i, 0)),
            pl.BlockSpec((_BM, 1), lambda i: (i, 0)),
        ],
        out_shape=[
            jax.ShapeDtypeStruct((NPAD, L), jnp.float32),
            jax.ShapeDtypeStruct((NPAD, L), jnp.float32),
            jax.ShapeDtypeStruct((NPAD, C), jnp.float32),
            jax.ShapeDtypeStruct((NPAD, F), jnp.float32),
            jax.ShapeDtypeStruct((NPAD, 1), jnp.float32),
        ],
    )(h, W_mu, b_mu.reshape(1, L),
      W_var, b_var.reshape(1, L), Wd1, bd1.reshape(1, -1),
      Wd2, bd2.reshape(1, -1), W_scale, b_scale.reshape(1, C),
      W_r, b_r.reshape(1, F), W_do, b_do.reshape(1, 1))


def kernel(x, edge_index, Ws0, Wn0, b0, Ws1, Wn1, b1, W_mu, b_mu, W_var,
           b_var, Wd1, bd1, Wd2, bd2, W_scale, b_scale, W_r, b_r, W_do, b_do):
    src = edge_index[0]
    dst = edge_index[1]
    x_pad = jnp.pad(x, ((0, NPAD - N), (0, 0)))
    sc_aggregate = _get_sc_aggregate()

    h0 = _tc_pre(x_pad)

    Wss = jnp.stack([Ws0.reshape(2, FH, 2 * FH), Ws1.reshape(2, FH, 2 * FH)])
    Wns = jnp.stack([Wn0.reshape(2, FH, 2 * FH), Wn1.reshape(2, FH, 2 * FH)])
    bs = jnp.stack([b0.reshape(1, 2 * FH), b1.reshape(1, 2 * FH)])

    def _scan_body(h, wk):
        Ws, Wn, b = wk
        sums, deg = sc_aggregate(h.reshape(2 * NPAD, FH), src, dst)
        h_new = _tc_layer(h, sums, deg.reshape(NPAD, 1), Ws, Wn, b)
        return h_new, None

    h2, _ = lax.scan(_scan_body, h0, (Wss, Wns, bs))
    z_loc, z_scale, px_scale, px_r, px_l = _tc_head(
        h2, W_mu, b_mu, W_var, b_var, Wd1, bd1, Wd2, bd2,
        W_scale, b_scale, W_r, b_r, W_do, b_do)
    return (z_loc[:N], z_scale[:N], px_scale[:N], px_r[:N], px_l[:N])


# speedup vs baseline: 6.1592x; 1.0684x over previous
import functools

import jax
import jax.numpy as jnp
from jax import lax
from jax.experimental import pallas as pl
from jax.experimental.pallas import tpu as pltpu
from jax.experimental.pallas import tpu_sc as plsc

N = 10000
NPAD = 10240
E = 160000
FH = 128
NSUB = 16
NCORE = 2
K = 128
EPT = E // NSUB
NCHUNK = EPT // K
TAIL = EPT - NCHUNK * K
NPR = NPAD // NSUB


@functools.cache
def _get_sc_aggregate():
    mesh = plsc.VectorSubcoreMesh(core_axis_name="c", subcore_axis_name="s",
                                  num_cores=NCORE, num_subcores=NSUB)

    @functools.partial(
        pl.kernel,
        out_type=[
            jax.ShapeDtypeStruct((NCORE, NPAD, FH), jnp.float32),
            jax.ShapeDtypeStruct((NPAD,), jnp.float32),
        ],
        mesh=mesh,
        scratch_types=[
            pltpu.VMEM((2, K), jnp.int32),
            pltpu.VMEM((2, K), jnp.int32),
            pltpu.VMEM((2, K, FH), jnp.float32),
            pltpu.VMEM((16,), jnp.int32),
            pltpu.VMEM((16,), jnp.int32),
            pltpu.VMEM((K,), jnp.float32),
            pltpu.VMEM((NPR,), jnp.float32),
            pltpu.VMEM_SHARED((NPAD, FH), jnp.float32),
            pltpu.VMEM_SHARED((NPAD,), jnp.float32),
        ] + [pltpu.SemaphoreType.DMA] * 8,
    )
    def _sc_aggregate(h_hbm, src_hbm, dst_hbm, sums_hbm, deg_hbm,
                      srcadj_v, dstb_v, rows_v, tsrc_v, tdst_v, ones_v,
                      zdeg_v, acc_sh, deg_sh, *sems):
        c = lax.axis_index("c")
        s = lax.axis_index("s")
        gsems = sems[0:2]
        ssems = sems[2:4]
        xsems = sems[4:6]
        dxsems = sems[6:8]

        def _zrow(i, carry):
            for j in range(FH // 16):
                rows_v[0, i, pl.ds(j * 16, 16)] = jnp.zeros((16,), jnp.float32)
            return carry
        lax.fori_loop(0, K, _zrow, 0)
        for j in range(K // 16):
            ones_v[pl.ds(j * 16, 16)] = jnp.ones((16,), jnp.float32)

        def _zdeg(i, carry):
            zdeg_v[pl.ds(i * 16, 16)] = jnp.zeros((16,), jnp.float32)
            return carry
        lax.fori_loop(0, NPR // 16, _zdeg, 0)

        for r in range(NPR // K):
            pltpu.sync_copy(rows_v.at[0],
                            acc_sh.at[pl.ds(s * NPR + r * K, K), :])
        pltpu.sync_copy(zdeg_v, deg_sh.at[pl.ds(s * NPR, NPR)])
        plsc.subcore_barrier()

        coff = c * NPAD

        def _adjust(b):
            for j in range(K // 16):
                srcadj_v[b, pl.ds(j * 16, 16)] = (
                    srcadj_v[b, pl.ds(j * 16, 16)] + coff)

        def _gather_start(b):
            pltpu.async_copy(h_hbm.at[srcadj_v.at[b]], rows_v.at[b], gsems[b])

        def _gather_wait(b):
            pltpu.make_async_copy(h_hbm.at[srcadj_v.at[b]], rows_v.at[b],
                                  gsems[b]).wait()

        def _srcidx_start(i, b):
            pltpu.async_copy(src_hbm.at[pl.ds(s * EPT + i * K, K)],
                             srcadj_v.at[b], xsems[b])

        def _srcidx_wait(i, b):
            pltpu.make_async_copy(src_hbm.at[pl.ds(s * EPT + i * K, K)],
                                  srcadj_v.at[b], xsems[b]).wait()

        def _dstidx_start(i, b):
            pltpu.async_copy(dst_hbm.at[pl.ds(s * EPT + i * K, K)],
                             dstb_v.at[b], dxsems[b])

        def _dstidx_wait(i, b):
            pltpu.make_async_copy(dst_hbm.at[pl.ds(s * EPT + i * K, K)],
                                  dstb_v.at[b], dxsems[b]).wait()

        for b in (0, 1):
            _srcidx_start(b, b)
            _dstidx_start(b, b)
            _srcidx_wait(b, b)
            _adjust(b)
            _gather_start(b)

        def _consume(i, b, nxt):
            _gather_wait(b)
            _dstidx_wait(i, b)
            sc = pltpu.async_copy(rows_v.at[b], acc_sh.at[dstb_v.at[b]],
                                  ssems[b], add=True)
            dg = pltpu.async_copy(ones_v, deg_sh.at[dstb_v.at[b]],
                                  ssems[b], add=True)
            if nxt:
                _srcidx_start(i + 2, b)
                _srcidx_wait(i + 2, b)
                _adjust(b)
            sc.wait()
            dg.wait()
            if nxt:
                _dstidx_start(i + 2, b)
                _gather_start(b)

        def _pair(p, carry):
            _consume(2 * p, 0, True)
            _consume(2 * p + 1, 1, True)
            return carry
        lax.fori_loop(0, NCHUNK // 2 - 1, _pair, 0)
        _consume(NCHUNK - 2, 0, False)
        _consume(NCHUNK - 1, 1, False)

        toff = s * EPT + NCHUNK * K
        pltpu.sync_copy(src_hbm.at[pl.ds(toff, TAIL)], tsrc_v)
        pltpu.sync_copy(dst_hbm.at[pl.ds(toff, TAIL)], tdst_v)
        tsrc_v[...] = tsrc_v[...] + coff
        pltpu.async_copy(h_hbm.at[tsrc_v], rows_v.at[0, pl.ds(0, TAIL)],
                         gsems[0]).wait()
        pltpu.sync_copy(rows_v.at[0, pl.ds(0, TAIL)], acc_sh.at[tdst_v],
                        add=True)
        pltpu.sync_copy(ones_v.at[pl.ds(0, TAIL)], deg_sh.at[tdst_v],
                        add=True)
        plsc.subcore_barrier()

        def _wb(r, carry):
            base = s * NPR + r * K
            pltpu.sync_copy(acc_sh.at[pl.ds(base, K), :], rows_v.at[0])
            pltpu.sync_copy(rows_v.at[0], sums_hbm.at[c, pl.ds(base, K), :])
            return carry
        lax.fori_loop(0, NPR // K, _wb, 0)

        @pl.when(c == 0)
        def _():
            pltpu.sync_copy(deg_sh.at[pl.ds(s * NPR, NPR)], zdeg_v)
            pltpu.sync_copy(zdeg_v, deg_hbm.at[pl.ds(s * NPR, NPR)])

    return _sc_aggregate


_BM = 512


def _pre_body(x_ref, out_ref):
    h = jnp.log(x_ref[...] + 1.0)
    out_ref[0] = h[:, :FH]
    out_ref[1] = h[:, FH:]


def _tc_pre(x_pad):
    return pl.pallas_call(
        _pre_body,
        grid=(NPAD // _BM,),
        in_specs=[pl.BlockSpec((_BM, 2 * FH), lambda i: (i, 0))],
        out_specs=pl.BlockSpec((2, _BM, FH), lambda i: (0, i, 0)),
        out_shape=jax.ShapeDtypeStruct((2, NPAD, FH), jnp.float32),
    )(x_pad)


def _dot(a, b):
    return jnp.dot(a, b, preferred_element_type=jnp.float32)


def _layer_math(h_ref, s_ref, deg_ref, Ws_ref, Wn_ref, b_ref):
    inv = 1.0 / jnp.maximum(deg_ref[...], 1.0)
    out = (_dot(h_ref[0], Ws_ref[0]) + _dot(h_ref[1], Ws_ref[1])
           + _dot(s_ref[0] * inv, Wn_ref[0]) + _dot(s_ref[1] * inv, Wn_ref[1])
           + b_ref[...])
    out = jnp.maximum(out, 0.0)
    nrm = jnp.sqrt(jnp.sum(out * out, axis=1, keepdims=True))
    return out / jnp.maximum(nrm, 1e-12)


def _layer_body(h_ref, s_ref, deg_ref, Ws_ref, Wn_ref, b_ref, out_ref):
    out = _layer_math(h_ref, s_ref, deg_ref, Ws_ref, Wn_ref, b_ref)
    out_ref[0] = out[:, :FH]
    out_ref[1] = out[:, FH:]


def _tc_layer(h, sums, deg2d, Ws, Wn, b):
    w_spec = pl.BlockSpec((2, FH, 2 * FH), lambda i: (0, 0, 0))
    return pl.pallas_call(
        _layer_body,
        grid=(NPAD // _BM,),
        in_specs=[
            pl.BlockSpec((2, _BM, FH), lambda i: (0, i, 0)),
            pl.BlockSpec((2, _BM, FH), lambda i: (0, i, 0)),
            pl.BlockSpec((_BM, 1), lambda i: (i, 0)),
            w_spec, w_spec,
            pl.BlockSpec((1, 2 * FH), lambda i: (0, 0)),
        ],
        out_specs=pl.BlockSpec((2, _BM, FH), lambda i: (0, i, 0)),
        out_shape=jax.ShapeDtypeStruct((2, NPAD, FH), jnp.float32),
    )(h, sums, deg2d, Ws, Wn, b)


def _head_body(h_ref,
               Wmu_ref, bmu_ref, Wvar_ref, bvar_ref,
               Wd1_ref, bd1_ref, Wd2_ref, bd2_ref,
               Wsc_ref, bsc_ref, Wr_ref, br_ref, Wdo_ref, bdo_ref,
               zloc_ref, zscale_ref, pxs_ref, pxr_ref, pxl_ref):
    h2 = jnp.concatenate([h_ref[0], h_ref[1]], axis=1)
    z_loc = _dot(h2, Wmu_ref[...]) + bmu_ref[...]
    zloc_ref[...] = z_loc
    zscale_ref[...] = jnp.exp(_dot(h2, Wvar_ref[...]) + bvar_ref[...]) + 1e-6
    px = jnp.maximum(_dot(z_loc, Wd1_ref[...]) + bd1_ref[...], 0.0)
    px = jnp.maximum(_dot(px, Wd2_ref[...]) + bd2_ref[...], 0.0)
    t = _dot(px, Wsc_ref[...]) + bsc_ref[...]
    t = t - jnp.max(t, axis=1, keepdims=True)
    e = jnp.exp(t)
    pxs_ref[...] = e / jnp.sum(e, axis=1, keepdims=True)
    pxr_ref[...] = _dot(px, Wr_ref[...]) + br_ref[...]
    pxl_ref[...] = _dot(px, Wdo_ref[...]) + bdo_ref[...]


def _full_spec(shape):
    nd = len(shape)
    return pl.BlockSpec(shape, lambda i, _nd=nd: (0,) * _nd)


def _tc_head(h, W_mu, b_mu, W_var, b_var,
             Wd1, bd1, Wd2, bd2, W_scale, b_scale, W_r, b_r, W_do, b_do):
    L = W_mu.shape[1]
    C = W_scale.shape[1]
    F = W_r.shape[1]
    return pl.pallas_call(
        _head_body,
        grid=(NPAD // _BM,),
        in_specs=[
            pl.BlockSpec((2, _BM, FH), lambda i: (0, i, 0)),
            _full_spec(W_mu.shape), _full_spec((1, L)),
            _full_spec(W_var.shape), _full_spec((1, L)),
            _full_spec(Wd1.shape), _full_spec((1, 2 * FH)),
            _full_spec(Wd2.shape), _full_spec((1, 2 * FH)),
            _full_spec(W_scale.shape), _full_spec((1, C)),
            _full_spec(W_r.shape), _full_spec((1, F)),
            _full_spec(W_do.shape), _full_spec((1, 1)),
        ],
        out_specs=[
            pl.BlockSpec((_BM, L), lambda i: (i, 0)),
            pl.BlockSpec((_BM, L), lambda i: (i, 0)),
            pl.BlockSpec((_BM, C), lambda i: (i, 0)),
            pl.BlockSpec((_BM, F), lambda i: (i, 0)),
            pl.BlockSpec((_BM, 1), lambda i: (i, 0)),
        ],
        out_shape=[
            jax.ShapeDtypeStruct((NPAD, L), jnp.float32),
            jax.ShapeDtypeStruct((NPAD, L), jnp.float32),
            jax.ShapeDtypeStruct((NPAD, C), jnp.float32),
            jax.ShapeDtypeStruct((NPAD, F), jnp.float32),
            jax.ShapeDtypeStruct((NPAD, 1), jnp.float32),
        ],
    )(h, W_mu, b_mu.reshape(1, L),
      W_var, b_var.reshape(1, L), Wd1, bd1.reshape(1, -1),
      Wd2, bd2.reshape(1, -1), W_scale, b_scale.reshape(1, C),
      W_r, b_r.reshape(1, F), W_do, b_do.reshape(1, 1))


def kernel(x, edge_index, Ws0, Wn0, b0, Ws1, Wn1, b1, W_mu, b_mu, W_var,
           b_var, Wd1, bd1, Wd2, bd2, W_scale, b_scale, W_r, b_r, W_do, b_do):
    src = edge_index[0]
    dst = edge_index[1]
    x_pad = jnp.pad(x, ((0, NPAD - N), (0, 0)))
    sc_aggregate = _get_sc_aggregate()

    h0 = _tc_pre(x_pad)

    Wss = jnp.stack([Ws0.reshape(2, FH, 2 * FH), Ws1.reshape(2, FH, 2 * FH)])
    Wns = jnp.stack([Wn0.reshape(2, FH, 2 * FH), Wn1.reshape(2, FH, 2 * FH)])
    bs = jnp.stack([b0.reshape(1, 2 * FH), b1.reshape(1, 2 * FH)])

    def _scan_body(h, wk):
        Ws, Wn, b = wk
        sums, deg = sc_aggregate(h.reshape(2 * NPAD, FH), src, dst)
        h_new = _tc_layer(h, sums, deg.reshape(NPAD, 1), Ws, Wn, b)
        return h_new, None

    h2, _ = lax.scan(_scan_body, h0, (Wss, Wns, bs))
    z_loc, z_scale, px_scale, px_r, px_l = _tc_head(
        h2, W_mu, b_mu, W_var, b_var, Wd1, bd1, Wd2, bd2,
        W_scale, b_scale, W_r, b_r, W_do, b_do)
    return (z_loc[:N], z_scale[:N], px_scale[:N], px_r[:N], px_l[:N])


# speedup vs baseline: 6.1640x; 1.0008x over previous
import functools

import jax
import jax.numpy as jnp
from jax import lax
from jax.experimental import pallas as pl
from jax.experimental.pallas import tpu as pltpu
from jax.experimental.pallas import tpu_sc as plsc

N = 10000
NPAD = 10240
E = 160000
FH = 128
NSUB = 16
NCORE = 2
K = 128
EPT = E // NSUB
NCHUNK = EPT // K
TAIL = EPT - NCHUNK * K
NPR = NPAD // NSUB


@functools.cache
def _get_sc_aggregate():
    mesh = plsc.VectorSubcoreMesh(core_axis_name="c", subcore_axis_name="s",
                                  num_cores=NCORE, num_subcores=NSUB)

    @functools.partial(
        pl.kernel,
        out_type=[
            jax.ShapeDtypeStruct((NCORE, NPAD, FH), jnp.float32),
            jax.ShapeDtypeStruct((NPAD,), jnp.float32),
        ],
        mesh=mesh,
        scratch_types=[
            pltpu.VMEM((2, K), jnp.int32),
            pltpu.VMEM((2, K), jnp.int32),
            pltpu.VMEM((2, K, FH), jnp.float32),
            pltpu.VMEM((16,), jnp.int32),
            pltpu.VMEM((16,), jnp.int32),
            pltpu.VMEM((K,), jnp.float32),
            pltpu.VMEM((NPR,), jnp.float32),
            pltpu.VMEM_SHARED((NPAD, FH), jnp.float32),
            pltpu.VMEM_SHARED((NPAD,), jnp.float32),
        ] + [pltpu.SemaphoreType.DMA] * 8,
    )
    def _sc_aggregate(h_hbm, src_hbm, dst_hbm, sums_hbm, deg_hbm,
                      srcadj_v, dstb_v, rows_v, tsrc_v, tdst_v, ones_v,
                      zdeg_v, acc_sh, deg_sh, *sems):
        c = lax.axis_index("c")
        s = lax.axis_index("s")
        gsems = sems[0:2]
        ssems = sems[2:4]
        xsems = sems[4:6]
        dxsems = sems[6:8]

        def _zrow(i, carry):
            for j in range(FH // 16):
                rows_v[0, i, pl.ds(j * 16, 16)] = jnp.zeros((16,), jnp.float32)
            return carry
        lax.fori_loop(0, K, _zrow, 0)
        for j in range(K // 16):
            ones_v[pl.ds(j * 16, 16)] = jnp.ones((16,), jnp.float32)

        def _zdeg(i, carry):
            zdeg_v[pl.ds(i * 16, 16)] = jnp.zeros((16,), jnp.float32)
            return carry
        lax.fori_loop(0, NPR // 16, _zdeg, 0)

        for r in range(NPR // K):
            pltpu.sync_copy(rows_v.at[0],
                            acc_sh.at[pl.ds(s * NPR + r * K, K), :])
        pltpu.sync_copy(zdeg_v, deg_sh.at[pl.ds(s * NPR, NPR)])
        plsc.subcore_barrier()

        coff = c * NPAD

        def _adjust(b):
            for j in range(K // 16):
                srcadj_v[b, pl.ds(j * 16, 16)] = (
                    srcadj_v[b, pl.ds(j * 16, 16)] + coff)

        def _gather_start(b):
            pltpu.async_copy(h_hbm.at[srcadj_v.at[b]], rows_v.at[b], gsems[b])

        def _gather_wait(b):
            pltpu.make_async_copy(h_hbm.at[srcadj_v.at[b]], rows_v.at[b],
                                  gsems[b]).wait()

        def _srcidx_start(i, b):
            pltpu.async_copy(src_hbm.at[pl.ds(s * EPT + i * K, K)],
                             srcadj_v.at[b], xsems[b])

        def _srcidx_wait(i, b):
            pltpu.make_async_copy(src_hbm.at[pl.ds(s * EPT + i * K, K)],
                                  srcadj_v.at[b], xsems[b]).wait()

        def _dstidx_start(i, b):
            pltpu.async_copy(dst_hbm.at[pl.ds(s * EPT + i * K, K)],
                             dstb_v.at[b], dxsems[b])

        def _dstidx_wait(i, b):
            pltpu.make_async_copy(dst_hbm.at[pl.ds(s * EPT + i * K, K)],
                                  dstb_v.at[b], dxsems[b]).wait()

        for b in (0, 1):
            _srcidx_start(b, b)
            _dstidx_start(b, b)
            _srcidx_wait(b, b)
            _adjust(b)
            _gather_start(b)

        def _consume(i, b, nxt):
            _gather_wait(b)
            _dstidx_wait(i, b)
            sc = pltpu.async_copy(rows_v.at[b], acc_sh.at[dstb_v.at[b]],
                                  ssems[b], add=True)
            dg = pltpu.async_copy(ones_v, deg_sh.at[dstb_v.at[b]],
                                  ssems[b], add=True)
            if nxt:
                _srcidx_start(i + 2, b)
                _srcidx_wait(i + 2, b)
                _adjust(b)
            sc.wait()
            dg.wait()
            if nxt:
                _dstidx_start(i + 2, b)
                _gather_start(b)

        def _pair(p, carry):
            _consume(2 * p, 0, True)
            _consume(2 * p + 1, 1, True)
            return carry
        lax.fori_loop(0, NCHUNK // 2 - 1, _pair, 0)
        _consume(NCHUNK - 2, 0, False)
        _consume(NCHUNK - 1, 1, False)

        toff = s * EPT + NCHUNK * K
        pltpu.sync_copy(src_hbm.at[pl.ds(toff, TAIL)], tsrc_v)
        pltpu.sync_copy(dst_hbm.at[pl.ds(toff, TAIL)], tdst_v)
        tsrc_v[...] = tsrc_v[...] + coff
        pltpu.async_copy(h_hbm.at[tsrc_v], rows_v.at[0, pl.ds(0, TAIL)],
                         gsems[0]).wait()
        pltpu.sync_copy(rows_v.at[0, pl.ds(0, TAIL)], acc_sh.at[tdst_v],
                        add=True)
        pltpu.sync_copy(ones_v.at[pl.ds(0, TAIL)], deg_sh.at[tdst_v],
                        add=True)
        plsc.subcore_barrier()

        def _wb(r, carry):
            base = s * NPR + r * K
            pltpu.sync_copy(acc_sh.at[pl.ds(base, K), :], rows_v.at[0])
            pltpu.sync_copy(rows_v.at[0], sums_hbm.at[c, pl.ds(base, K), :])
            return carry
        lax.fori_loop(0, NPR // K, _wb, 0)

        @pl.when(c == 0)
        def _():
            pltpu.sync_copy(deg_sh.at[pl.ds(s * NPR, NPR)], zdeg_v)
            pltpu.sync_copy(zdeg_v, deg_hbm.at[pl.ds(s * NPR, NPR)])

    return _sc_aggregate


_BM = 512


def _pre_body(x_ref, out_ref):
    h = jnp.log(x_ref[...] + 1.0)
    out_ref[0] = h[:, :FH]
    out_ref[1] = h[:, FH:]


def _tc_pre(x_pad):
    return pl.pallas_call(
        _pre_body,
        grid=(NPAD // _BM,),
        in_specs=[pl.BlockSpec((_BM, 2 * FH), lambda i: (i, 0))],
        out_specs=pl.BlockSpec((2, _BM, FH), lambda i: (0, i, 0)),
        out_shape=jax.ShapeDtypeStruct((2, NPAD, FH), jnp.float32),
    )(x_pad)


def _dot(a, b):
    return jnp.dot(a, b, preferred_element_type=jnp.float32)


def _self_body(h_ref, Ws_ref, b_ref, out_ref):
    out = _dot(h_ref[0], Ws_ref[0]) + _dot(h_ref[1], Ws_ref[1]) + b_ref[...]
    out_ref[0] = out[:, :FH]
    out_ref[1] = out[:, FH:]


def _tc_self(h, Ws, b):
    return pl.pallas_call(
        _self_body,
        grid=(NPAD // _BM,),
        in_specs=[
            pl.BlockSpec((2, _BM, FH), lambda i: (0, i, 0)),
            pl.BlockSpec((2, FH, 2 * FH), lambda i: (0, 0, 0)),
            pl.BlockSpec((1, 2 * FH), lambda i: (0, 0)),
        ],
        out_specs=pl.BlockSpec((2, _BM, FH), lambda i: (0, i, 0)),
        out_shape=jax.ShapeDtypeStruct((2, NPAD, FH), jnp.float32),
    )(h, Ws, b)


def _combine_body(sf_ref, s_ref, deg_ref, Wn_ref, out_ref):
    inv = 1.0 / jnp.maximum(deg_ref[...], 1.0)
    out = (jnp.concatenate([sf_ref[0], sf_ref[1]], axis=1)
           + _dot(s_ref[0] * inv, Wn_ref[0]) + _dot(s_ref[1] * inv, Wn_ref[1]))
    out = jnp.maximum(out, 0.0)
    nrm = jnp.sqrt(jnp.sum(out * out, axis=1, keepdims=True))
    out = out / jnp.maximum(nrm, 1e-12)
    out_ref[0] = out[:, :FH]
    out_ref[1] = out[:, FH:]


def _tc_combine(sf, sums, deg2d, Wn):
    return pl.pallas_call(
        _combine_body,
        grid=(NPAD // _BM,),
        in_specs=[
            pl.BlockSpec((2, _BM, FH), lambda i: (0, i, 0)),
            pl.BlockSpec((2, _BM, FH), lambda i: (0, i, 0)),
            pl.BlockSpec((_BM, 1), lambda i: (i, 0)),
            pl.BlockSpec((2, FH, 2 * FH), lambda i: (0, 0, 0)),
        ],
        out_specs=pl.BlockSpec((2, _BM, FH), lambda i: (0, i, 0)),
        out_shape=jax.ShapeDtypeStruct((2, NPAD, FH), jnp.float32),
    )(sf, sums, deg2d, Wn)


def _head_body(h_ref,
               Wmv_ref, bmv_ref,
               Wd1_ref, bd1_ref, Wd2_ref, bd2_ref,
               Wr_ref, br_ref, Wsd_ref, bsd_ref,
               zloc_ref, zscale_ref, pxs_ref, pxr_ref, pxl_ref):
    L = zloc_ref.shape[1]
    C = pxs_ref.shape[1]
    h2 = jnp.concatenate([h_ref[0], h_ref[1]], axis=1)
    z = _dot(h2, Wmv_ref[...]) + bmv_ref[...]
    z_loc = z[:, :L]
    zloc_ref[...] = z_loc
    zscale_ref[...] = jnp.exp(z[:, L:]) + 1e-6
    px = jnp.maximum(_dot(z_loc, Wd1_ref[...]) + bd1_ref[...], 0.0)
    px = jnp.maximum(_dot(px, Wd2_ref[...]) + bd2_ref[...], 0.0)
    pxr_ref[...] = _dot(px, Wr_ref[...]) + br_ref[...]
    t = _dot(px, Wsd_ref[...]) + bsd_ref[...]
    ts = t[:, :C] - jnp.max(t[:, :C], axis=1, keepdims=True)
    e = jnp.exp(ts)
    pxs_ref[...] = e / jnp.sum(e, axis=1, keepdims=True)
    pxl_ref[...] = t[:, C:C + 1]


def _full_spec(shape):
    nd = len(shape)
    return pl.BlockSpec(shape, lambda i, _nd=nd: (0,) * _nd)


def _tc_head(h, W_mu, b_mu, W_var, b_var,
             Wd1, bd1, Wd2, bd2, W_scale, b_scale, W_r, b_r, W_do, b_do):
    L = W_mu.shape[1]
    C = W_scale.shape[1]
    F = W_r.shape[1]
    Wmv = jnp.concatenate([W_mu, W_var], axis=1)
    bmv = jnp.concatenate([b_mu, b_var]).reshape(1, 2 * L)
    Wsd = jnp.concatenate([W_scale, W_do], axis=1)
    bsd = jnp.concatenate([b_scale, b_do]).reshape(1, C + 1)
    return pl.pallas_call(
        _head_body,
        grid=(NPAD // _BM,),
        in_specs=[
            pl.BlockSpec((2, _BM, FH), lambda i: (0, i, 0)),
            _full_spec(Wmv.shape), _full_spec((1, 2 * L)),
            _full_spec(Wd1.shape), _full_spec((1, 2 * FH)),
            _full_spec(Wd2.shape), _full_spec((1, 2 * FH)),
            _full_spec(W_r.shape), _full_spec((1, F)),
            _full_spec(Wsd.shape), _full_spec((1, C + 1)),
        ],
        out_specs=[
            pl.BlockSpec((_BM, L), lambda i: (i, 0)),
            pl.BlockSpec((_BM, L), lambda i: (i, 0)),
            pl.BlockSpec((_BM, C), lambda i: (i, 0)),
            pl.BlockSpec((_BM, F), lambda i: (i, 0)),
            pl.BlockSpec((_BM, 1), lambda i: (i, 0)),
        ],
        out_shape=[
            jax.ShapeDtypeStruct((NPAD, L), jnp.float32),
            jax.ShapeDtypeStruct((NPAD, L), jnp.float32),
            jax.ShapeDtypeStruct((NPAD, C), jnp.float32),
            jax.ShapeDtypeStruct((NPAD, F), jnp.float32),
            jax.ShapeDtypeStruct((NPAD, 1), jnp.float32),
        ],
    )(h, Wmv, bmv, Wd1, bd1.reshape(1, -1),
      Wd2, bd2.reshape(1, -1), W_r, b_r.reshape(1, F), Wsd, bsd)


def kernel(x, edge_index, Ws0, Wn0, b0, Ws1, Wn1, b1, W_mu, b_mu, W_var,
           b_var, Wd1, bd1, Wd2, bd2, W_scale, b_scale, W_r, b_r, W_do, b_do):
    src = edge_index[0]
    dst = edge_index[1]
    x_pad = jnp.pad(x, ((0, NPAD - N), (0, 0)))
    sc_aggregate = _get_sc_aggregate()

    h0 = _tc_pre(x_pad)

    Wss = jnp.stack([Ws0.reshape(2, FH, 2 * FH), Ws1.reshape(2, FH, 2 * FH)])
    Wns = jnp.stack([Wn0.reshape(2, FH, 2 * FH), Wn1.reshape(2, FH, 2 * FH)])
    bs = jnp.stack([b0.reshape(1, 2 * FH), b1.reshape(1, 2 * FH)])

    def _scan_body(h, wk):
        Ws, Wn, b = wk
        sums, deg = sc_aggregate(h.reshape(2 * NPAD, FH), src, dst)
        sf = _tc_self(h, Ws, b)
        h_new = _tc_combine(sf, sums, deg.reshape(NPAD, 1), Wn)
        return h_new, None

    h2, _ = lax.scan(_scan_body, h0, (Wss, Wns, bs))
    z_loc, z_scale, px_scale, px_r, px_l = _tc_head(
        h2, W_mu, b_mu, W_var, b_var, Wd1, bd1, Wd2, bd2,
        W_scale, b_scale, W_r, b_r, W_do, b_do)
    return (z_loc[:N], z_scale[:N], px_scale[:N], px_r[:N], px_l[:N])


# speedup vs baseline: 6.5283x; 1.0591x over previous
import functools

import jax
import jax.numpy as jnp
from jax import lax
from jax.experimental import pallas as pl
from jax.experimental.pallas import tpu as pltpu
from jax.experimental.pallas import tpu_sc as plsc

N = 10000
NPAD = 10240
E = 160000
FH = 128
NSUB = 16
NCORE = 2
K = 128
EPT = E // NSUB
NCHUNK = EPT // K
TAIL = EPT - NCHUNK * K
NPR = NPAD // NSUB


@functools.cache
def _get_sc_aggregate():
    mesh = plsc.VectorSubcoreMesh(core_axis_name="c", subcore_axis_name="s",
                                  num_cores=NCORE, num_subcores=NSUB)

    @functools.partial(
        pl.kernel,
        out_type=[
            jax.ShapeDtypeStruct((NCORE, NPAD, FH), jnp.float32),
            jax.ShapeDtypeStruct((NPAD,), jnp.float32),
        ],
        mesh=mesh,
        scratch_types=[
            pltpu.VMEM((2, K), jnp.int32),
            pltpu.VMEM((2, K), jnp.int32),
            pltpu.VMEM((2, K, FH), jnp.float32),
            pltpu.VMEM((16,), jnp.int32),
            pltpu.VMEM((16,), jnp.int32),
            pltpu.VMEM((K,), jnp.float32),
            pltpu.VMEM((NPR,), jnp.float32),
            pltpu.VMEM_SHARED((NPAD, FH), jnp.float32),
            pltpu.VMEM_SHARED((NPAD,), jnp.float32),
        ] + [pltpu.SemaphoreType.DMA] * 8,
    )
    def _sc_aggregate(h_hbm, src_hbm, dst_hbm, sums_hbm, deg_hbm,
                      srcadj_v, dstb_v, rows_v, tsrc_v, tdst_v, ones_v,
                      zdeg_v, acc_sh, deg_sh, *sems):
        c = lax.axis_index("c")
        s = lax.axis_index("s")
        gsems = sems[0:2]
        ssems = sems[2:4]
        xsems = sems[4:6]
        dxsems = sems[6:8]

        def _zrow(i, carry):
            for j in range(FH // 16):
                rows_v[0, i, pl.ds(j * 16, 16)] = jnp.zeros((16,), jnp.float32)
            return carry
        lax.fori_loop(0, K, _zrow, 0)
        for j in range(K // 16):
            ones_v[pl.ds(j * 16, 16)] = jnp.ones((16,), jnp.float32)

        def _zdeg(i, carry):
            zdeg_v[pl.ds(i * 16, 16)] = jnp.zeros((16,), jnp.float32)
            return carry
        lax.fori_loop(0, NPR // 16, _zdeg, 0)

        for r in range(NPR // K):
            pltpu.async_copy(rows_v.at[0],
                             acc_sh.at[pl.ds(s * NPR + r * K, K), :],
                             gsems[0])
        for r in range(NPR // K):
            pltpu.make_async_copy(rows_v.at[0],
                                  acc_sh.at[pl.ds(s * NPR + r * K, K), :],
                                  gsems[0]).wait()
        pltpu.sync_copy(zdeg_v, deg_sh.at[pl.ds(s * NPR, NPR)])
        plsc.subcore_barrier()

        coff = c * NPAD

        def _adjust(b):
            for j in range(K // 16):
                srcadj_v[b, pl.ds(j * 16, 16)] = (
                    srcadj_v[b, pl.ds(j * 16, 16)] + coff)

        def _gather_start(b):
            pltpu.async_copy(h_hbm.at[srcadj_v.at[b]], rows_v.at[b], gsems[b])

        def _gather_wait(b):
            pltpu.make_async_copy(h_hbm.at[srcadj_v.at[b]], rows_v.at[b],
                                  gsems[b]).wait()

        def _srcidx_start(i, b):
            pltpu.async_copy(src_hbm.at[pl.ds(s * EPT + i * K, K)],
                             srcadj_v.at[b], xsems[b])

        def _srcidx_wait(i, b):
            pltpu.make_async_copy(src_hbm.at[pl.ds(s * EPT + i * K, K)],
                                  srcadj_v.at[b], xsems[b]).wait()

        def _dstidx_start(i, b):
            pltpu.async_copy(dst_hbm.at[pl.ds(s * EPT + i * K, K)],
                             dstb_v.at[b], dxsems[b])

        def _dstidx_wait(i, b):
            pltpu.make_async_copy(dst_hbm.at[pl.ds(s * EPT + i * K, K)],
                                  dstb_v.at[b], dxsems[b]).wait()

        for b in (0, 1):
            _srcidx_start(b, b)
            _dstidx_start(b, b)
            _srcidx_wait(b, b)
            _adjust(b)
            _gather_start(b)

        def _consume(i, b, nxt):
            _gather_wait(b)
            _dstidx_wait(i, b)
            sc = pltpu.async_copy(rows_v.at[b], acc_sh.at[dstb_v.at[b]],
                                  ssems[b], add=True)
            dg = pltpu.async_copy(ones_v, deg_sh.at[dstb_v.at[b]],
                                  ssems[b], add=True)
            if nxt:
                _srcidx_start(i + 2, b)
                _srcidx_wait(i + 2, b)
                _adjust(b)
            sc.wait()
            dg.wait()
            if nxt:
                _dstidx_start(i + 2, b)
                _gather_start(b)

        def _pair(p, carry):
            _consume(2 * p, 0, True)
            _consume(2 * p + 1, 1, True)
            return carry
        lax.fori_loop(0, NCHUNK // 2 - 1, _pair, 0)
        _consume(NCHUNK - 2, 0, False)
        _consume(NCHUNK - 1, 1, False)

        toff = s * EPT + NCHUNK * K
        pltpu.sync_copy(src_hbm.at[pl.ds(toff, TAIL)], tsrc_v)
        pltpu.sync_copy(dst_hbm.at[pl.ds(toff, TAIL)], tdst_v)
        tsrc_v[...] = tsrc_v[...] + coff
        pltpu.async_copy(h_hbm.at[tsrc_v], rows_v.at[0, pl.ds(0, TAIL)],
                         gsems[0]).wait()
        pltpu.sync_copy(rows_v.at[0, pl.ds(0, TAIL)], acc_sh.at[tdst_v],
                        add=True)
        pltpu.sync_copy(ones_v.at[pl.ds(0, TAIL)], deg_sh.at[tdst_v],
                        add=True)
        plsc.subcore_barrier()

        def _wb_out(r, b):
            return pltpu.make_async_copy(
                rows_v.at[b],
                sums_hbm.at[c, pl.ds(s * NPR + r * K, K), :], ssems[b])

        for r in range(NPR // K):
            b = r % 2
            if r >= 2:
                _wb_out(r - 2, b).wait()
            pltpu.async_copy(acc_sh.at[pl.ds(s * NPR + r * K, K), :],
                             rows_v.at[b], gsems[b]).wait()
            pltpu.async_copy(rows_v.at[b],
                             sums_hbm.at[c, pl.ds(s * NPR + r * K, K), :],
                             ssems[b])
        _wb_out(NPR // K - 2, (NPR // K - 2) % 2).wait()
        _wb_out(NPR // K - 1, (NPR // K - 1) % 2).wait()

        @pl.when(c == 0)
        def _():
            pltpu.sync_copy(deg_sh.at[pl.ds(s * NPR, NPR)], zdeg_v)
            pltpu.sync_copy(zdeg_v, deg_hbm.at[pl.ds(s * NPR, NPR)])

    return _sc_aggregate


_BM = 512


def _pre_body(x_ref, out_ref):
    h = jnp.log(x_ref[...] + 1.0)
    out_ref[0] = h[:, :FH]
    out_ref[1] = h[:, FH:]


def _tc_pre(x_pad):
    return pl.pallas_call(
        _pre_body,
        grid=(NPAD // _BM,),
        in_specs=[pl.BlockSpec((_BM, 2 * FH), lambda i: (i, 0))],
        out_specs=pl.BlockSpec((2, _BM, FH), lambda i: (0, i, 0)),
        out_shape=jax.ShapeDtypeStruct((2, NPAD, FH), jnp.float32),
    )(x_pad)


def _dot(a, b):
    return jnp.dot(a, b, preferred_element_type=jnp.float32)


def _self_body(h_ref, Ws_ref, b_ref, out_ref):
    out = _dot(h_ref[0], Ws_ref[0]) + _dot(h_ref[1], Ws_ref[1]) + b_ref[...]
    out_ref[0] = out[:, :FH]
    out_ref[1] = out[:, FH:]


def _tc_self(h, Ws, b):
    return pl.pallas_call(
        _self_body,
        grid=(NPAD // _BM,),
        in_specs=[
            pl.BlockSpec((2, _BM, FH), lambda i: (0, i, 0)),
            pl.BlockSpec((2, FH, 2 * FH), lambda i: (0, 0, 0)),
            pl.BlockSpec((1, 2 * FH), lambda i: (0, 0)),
        ],
        out_specs=pl.BlockSpec((2, _BM, FH), lambda i: (0, i, 0)),
        out_shape=jax.ShapeDtypeStruct((2, NPAD, FH), jnp.float32),
    )(h, Ws, b)


def _combine_body(sf_ref, s_ref, deg_ref, Wn_ref, out_ref):
    inv = 1.0 / jnp.maximum(deg_ref[...], 1.0)
    out = (jnp.concatenate([sf_ref[0], sf_ref[1]], axis=1)
           + _dot(s_ref[0] * inv, Wn_ref[0]) + _dot(s_ref[1] * inv, Wn_ref[1]))
    out = jnp.maximum(out, 0.0)
    nrm = jnp.sqrt(jnp.sum(out * out, axis=1, keepdims=True))
    out = out / jnp.maximum(nrm, 1e-12)
    out_ref[0] = out[:, :FH]
    out_ref[1] = out[:, FH:]


def _tc_combine(sf, sums, deg2d, Wn):
    return pl.pallas_call(
        _combine_body,
        grid=(NPAD // _BM,),
        in_specs=[
            pl.BlockSpec((2, _BM, FH), lambda i: (0, i, 0)),
            pl.BlockSpec((2, _BM, FH), lambda i: (0, i, 0)),
            pl.BlockSpec((_BM, 1), lambda i: (i, 0)),
            pl.BlockSpec((2, FH, 2 * FH), lambda i: (0, 0, 0)),
        ],
        out_specs=pl.BlockSpec((2, _BM, FH), lambda i: (0, i, 0)),
        out_shape=jax.ShapeDtypeStruct((2, NPAD, FH), jnp.float32),
    )(sf, sums, deg2d, Wn)


def _head_body(h_ref,
               Wmv_ref, bmv_ref,
               Wd1_ref, bd1_ref, Wd2_ref, bd2_ref,
               Wr_ref, br_ref, Wsd_ref, bsd_ref,
               zloc_ref, zscale_ref, pxs_ref, pxr_ref, pxl_ref):
    L = zloc_ref.shape[1]
    C = pxs_ref.shape[1]
    h2 = jnp.concatenate([h_ref[0], h_ref[1]], axis=1)
    z = _dot(h2, Wmv_ref[...]) + bmv_ref[...]
    z_loc = z[:, :L]
    zloc_ref[...] = z_loc
    zscale_ref[...] = jnp.exp(z[:, L:]) + 1e-6
    px = jnp.maximum(_dot(z_loc, Wd1_ref[...]) + bd1_ref[...], 0.0)
    px = jnp.maximum(_dot(px, Wd2_ref[...]) + bd2_ref[...], 0.0)
    pxr_ref[...] = _dot(px, Wr_ref[...]) + br_ref[...]
    t = _dot(px, Wsd_ref[...]) + bsd_ref[...]
    ts = t[:, :C] - jnp.max(t[:, :C], axis=1, keepdims=True)
    e = jnp.exp(ts)
    pxs_ref[...] = e / jnp.sum(e, axis=1, keepdims=True)
    pxl_ref[...] = t[:, C:C + 1]


def _full_spec(shape):
    nd = len(shape)
    return pl.BlockSpec(shape, lambda i, _nd=nd: (0,) * _nd)


def _tc_head(h, W_mu, b_mu, W_var, b_var,
             Wd1, bd1, Wd2, bd2, W_scale, b_scale, W_r, b_r, W_do, b_do):
    L = W_mu.shape[1]
    C = W_scale.shape[1]
    F = W_r.shape[1]
    Wmv = jnp.concatenate([W_mu, W_var], axis=1)
    bmv = jnp.concatenate([b_mu, b_var]).reshape(1, 2 * L)
    Wsd = jnp.concatenate([W_scale, W_do], axis=1)
    bsd = jnp.concatenate([b_scale, b_do]).reshape(1, C + 1)
    return pl.pallas_call(
        _head_body,
        grid=(NPAD // _BM,),
        in_specs=[
            pl.BlockSpec((2, _BM, FH), lambda i: (0, i, 0)),
            _full_spec(Wmv.shape), _full_spec((1, 2 * L)),
            _full_spec(Wd1.shape), _full_spec((1, 2 * FH)),
            _full_spec(Wd2.shape), _full_spec((1, 2 * FH)),
            _full_spec(W_r.shape), _full_spec((1, F)),
            _full_spec(Wsd.shape), _full_spec((1, C + 1)),
        ],
        out_specs=[
            pl.BlockSpec((_BM, L), lambda i: (i, 0)),
            pl.BlockSpec((_BM, L), lambda i: (i, 0)),
            pl.BlockSpec((_BM, C), lambda i: (i, 0)),
            pl.BlockSpec((_BM, F), lambda i: (i, 0)),
            pl.BlockSpec((_BM, 1), lambda i: (i, 0)),
        ],
        out_shape=[
            jax.ShapeDtypeStruct((NPAD, L), jnp.float32),
            jax.ShapeDtypeStruct((NPAD, L), jnp.float32),
            jax.ShapeDtypeStruct((NPAD, C), jnp.float32),
            jax.ShapeDtypeStruct((NPAD, F), jnp.float32),
            jax.ShapeDtypeStruct((NPAD, 1), jnp.float32),
        ],
    )(h, Wmv, bmv, Wd1, bd1.reshape(1, -1),
      Wd2, bd2.reshape(1, -1), W_r, b_r.reshape(1, F), Wsd, bsd)


def kernel(x, edge_index, Ws0, Wn0, b0, Ws1, Wn1, b1, W_mu, b_mu, W_var,
           b_var, Wd1, bd1, Wd2, bd2, W_scale, b_scale, W_r, b_r, W_do, b_do):
    src = edge_index[0]
    dst = edge_index[1]
    x_pad = jnp.pad(x, ((0, NPAD - N), (0, 0)))
    sc_aggregate = _get_sc_aggregate()

    h0 = _tc_pre(x_pad)

    Wss = jnp.stack([Ws0.reshape(2, FH, 2 * FH), Ws1.reshape(2, FH, 2 * FH)])
    Wns = jnp.stack([Wn0.reshape(2, FH, 2 * FH), Wn1.reshape(2, FH, 2 * FH)])
    bs = jnp.stack([b0.reshape(1, 2 * FH), b1.reshape(1, 2 * FH)])

    def _scan_body(h, wk):
        Ws, Wn, b = wk
        sums, deg = sc_aggregate(h.reshape(2 * NPAD, FH), src, dst)
        sf = _tc_self(h, Ws, b)
        h_new = _tc_combine(sf, sums, deg.reshape(NPAD, 1), Wn)
        return h_new, None

    h2, _ = lax.scan(_scan_body, h0, (Wss, Wns, bs))
    z_loc, z_scale, px_scale, px_r, px_l = _tc_head(
        h2, W_mu, b_mu, W_var, b_var, Wd1, bd1, Wd2, bd2,
        W_scale, b_scale, W_r, b_r, W_do, b_do)
    return (z_loc[:N], z_scale[:N], px_scale[:N], px_r[:N], px_l[:N])


# speedup vs baseline: 7.0417x; 1.0786x over previous
import functools

import jax
import jax.numpy as jnp
from jax import lax
from jax.experimental import pallas as pl
from jax.experimental.pallas import tpu as pltpu
from jax.experimental.pallas import tpu_sc as plsc

N = 10000
NPAD = 10240
E = 160000
FH = 128
NSUB = 16
NCORE = 2
K = 128
EPT = E // NSUB
NCHUNK = EPT // K
TAIL = EPT - NCHUNK * K
NPR = NPAD // NSUB


@functools.cache
def _get_sc_aggregate():
    mesh = plsc.VectorSubcoreMesh(core_axis_name="c", subcore_axis_name="s",
                                  num_cores=NCORE, num_subcores=NSUB)

    @functools.partial(
        pl.kernel,
        out_type=[
            jax.ShapeDtypeStruct((NCORE, NPAD, FH), jnp.float32),
            jax.ShapeDtypeStruct((NPAD,), jnp.float32),
        ],
        mesh=mesh,
        scratch_types=[
            pltpu.VMEM((2, K), jnp.int32),
            pltpu.VMEM((2, K), jnp.int32),
            pltpu.VMEM((2, K, FH), jnp.float32),
            pltpu.VMEM((16,), jnp.int32),
            pltpu.VMEM((16,), jnp.int32),
            pltpu.VMEM((K,), jnp.float32),
            pltpu.VMEM((NPR,), jnp.float32),
            pltpu.VMEM_SHARED((NPAD, FH), jnp.float32),
            pltpu.VMEM_SHARED((NPAD,), jnp.float32),
        ] + [pltpu.SemaphoreType.DMA] * 8,
    )
    def _sc_aggregate(h_hbm, src_hbm, dst_hbm, sums_hbm, deg_hbm,
                      srcadj_v, dstb_v, rows_v, tsrc_v, tdst_v, ones_v,
                      zdeg_v, acc_sh, deg_sh, *sems):
        c = lax.axis_index("c")
        s = lax.axis_index("s")
        gsems = sems[0:2]
        ssems = sems[2:4]
        xsems = sems[4:6]
        dxsems = sems[6:8]

        def _zrow(i, carry):
            for j in range(FH // 16):
                rows_v[0, i, pl.ds(j * 16, 16)] = jnp.zeros((16,), jnp.float32)
            return carry
        lax.fori_loop(0, K, _zrow, 0)
        for j in range(K // 16):
            ones_v[pl.ds(j * 16, 16)] = jnp.ones((16,), jnp.float32)

        def _zdeg(i, carry):
            zdeg_v[pl.ds(i * 16, 16)] = jnp.zeros((16,), jnp.float32)
            return carry
        lax.fori_loop(0, NPR // 16, _zdeg, 0)

        for r in range(NPR // K):
            pltpu.async_copy(rows_v.at[0],
                             acc_sh.at[pl.ds(s * NPR + r * K, K), :],
                             gsems[0])
        for r in range(NPR // K):
            pltpu.make_async_copy(rows_v.at[0],
                                  acc_sh.at[pl.ds(s * NPR + r * K, K), :],
                                  gsems[0]).wait()
        pltpu.sync_copy(zdeg_v, deg_sh.at[pl.ds(s * NPR, NPR)])
        plsc.subcore_barrier()

        coff = c * NPAD

        def _adjust(b):
            for j in range(K // 16):
                srcadj_v[b, pl.ds(j * 16, 16)] = (
                    srcadj_v[b, pl.ds(j * 16, 16)] + coff)

        def _gather_start(b):
            pltpu.async_copy(h_hbm.at[srcadj_v.at[b]], rows_v.at[b], gsems[b])

        def _gather_wait(b):
            pltpu.make_async_copy(h_hbm.at[srcadj_v.at[b]], rows_v.at[b],
                                  gsems[b]).wait()

        def _srcidx_start(i, b):
            pltpu.async_copy(src_hbm.at[pl.ds(s * EPT + i * K, K)],
                             srcadj_v.at[b], xsems[b])

        def _srcidx_wait(i, b):
            pltpu.make_async_copy(src_hbm.at[pl.ds(s * EPT + i * K, K)],
                                  srcadj_v.at[b], xsems[b]).wait()

        def _dstidx_start(i, b):
            pltpu.async_copy(dst_hbm.at[pl.ds(s * EPT + i * K, K)],
                             dstb_v.at[b], dxsems[b])

        def _dstidx_wait(i, b):
            pltpu.make_async_copy(dst_hbm.at[pl.ds(s * EPT + i * K, K)],
                                  dstb_v.at[b], dxsems[b]).wait()

        for b in (0, 1):
            _srcidx_start(b, b)
            _dstidx_start(b, b)
            _srcidx_wait(b, b)
            _adjust(b)
            _gather_start(b)

        def _consume(i, b, nxt):
            _gather_wait(b)
            _dstidx_wait(i, b)
            sc = pltpu.async_copy(rows_v.at[b], acc_sh.at[dstb_v.at[b]],
                                  ssems[b], add=True)
            dg = pltpu.async_copy(ones_v, deg_sh.at[dstb_v.at[b]],
                                  ssems[b], add=True)
            if nxt:
                _srcidx_start(i + 2, b)
                _srcidx_wait(i + 2, b)
                _adjust(b)
            sc.wait()
            dg.wait()
            if nxt:
                _dstidx_start(i + 2, b)
                _gather_start(b)

        def _pair(p, carry):
            _consume(2 * p, 0, True)
            _consume(2 * p + 1, 1, True)
            return carry
        lax.fori_loop(0, NCHUNK // 2 - 1, _pair, 0)
        _consume(NCHUNK - 2, 0, False)
        _consume(NCHUNK - 1, 1, False)

        toff = s * EPT + NCHUNK * K
        pltpu.sync_copy(src_hbm.at[pl.ds(toff, TAIL)], tsrc_v)
        pltpu.sync_copy(dst_hbm.at[pl.ds(toff, TAIL)], tdst_v)
        tsrc_v[...] = tsrc_v[...] + coff
        pltpu.async_copy(h_hbm.at[tsrc_v], rows_v.at[0, pl.ds(0, TAIL)],
                         gsems[0]).wait()
        pltpu.sync_copy(rows_v.at[0, pl.ds(0, TAIL)], acc_sh.at[tdst_v],
                        add=True)
        pltpu.sync_copy(ones_v.at[pl.ds(0, TAIL)], deg_sh.at[tdst_v],
                        add=True)
        plsc.subcore_barrier()

        def _wb_out(r, b):
            return pltpu.make_async_copy(
                rows_v.at[b],
                sums_hbm.at[c, pl.ds(s * NPR + r * K, K), :], ssems[b])

        for r in range(NPR // K):
            b = r % 2
            if r >= 2:
                _wb_out(r - 2, b).wait()
            pltpu.async_copy(acc_sh.at[pl.ds(s * NPR + r * K, K), :],
                             rows_v.at[b], gsems[b]).wait()
            pltpu.async_copy(rows_v.at[b],
                             sums_hbm.at[c, pl.ds(s * NPR + r * K, K), :],
                             ssems[b])
        _wb_out(NPR // K - 2, (NPR // K - 2) % 2).wait()
        _wb_out(NPR // K - 1, (NPR // K - 1) % 2).wait()

        @pl.when(c == 0)
        def _():
            pltpu.sync_copy(deg_sh.at[pl.ds(s * NPR, NPR)], zdeg_v)
            pltpu.sync_copy(zdeg_v, deg_hbm.at[pl.ds(s * NPR, NPR)])

    return _sc_aggregate


_BM = 2048


def _pre_body(x_ref, out_ref):
    h = jnp.log(x_ref[...] + 1.0)
    out_ref[0] = h[:, :FH]
    out_ref[1] = h[:, FH:]


def _tc_pre(x):
    return pl.pallas_call(
        _pre_body,
        grid=(NPAD // _BM,),
        in_specs=[pl.BlockSpec((_BM, 2 * FH), lambda i: (i, 0))],
        out_specs=pl.BlockSpec((2, _BM, FH), lambda i: (0, i, 0)),
        out_shape=jax.ShapeDtypeStruct((2, NPAD, FH), jnp.float32),
    )(x)


def _dot(a, b):
    return jnp.dot(a, b, preferred_element_type=jnp.float32)


def _self_body(h_ref, Ws_ref, b_ref, out_ref):
    out = _dot(h_ref[0], Ws_ref[0]) + _dot(h_ref[1], Ws_ref[1]) + b_ref[...]
    out_ref[0] = out[:, :FH]
    out_ref[1] = out[:, FH:]


def _tc_self(h, Ws, b):
    return pl.pallas_call(
        _self_body,
        grid=(NPAD // _BM,),
        in_specs=[
            pl.BlockSpec((2, _BM, FH), lambda i: (0, i, 0)),
            pl.BlockSpec((2, FH, 2 * FH), lambda i: (0, 0, 0)),
            pl.BlockSpec((1, 2 * FH), lambda i: (0, 0)),
        ],
        out_specs=pl.BlockSpec((2, _BM, FH), lambda i: (0, i, 0)),
        out_shape=jax.ShapeDtypeStruct((2, NPAD, FH), jnp.float32),
    )(h, Ws, b)


def _combine_body(sf_ref, s_ref, deg_ref, Wn_ref, out_ref):
    inv = 1.0 / jnp.maximum(deg_ref[...], 1.0)
    out = (jnp.concatenate([sf_ref[0], sf_ref[1]], axis=1)
           + _dot(s_ref[0] * inv, Wn_ref[0]) + _dot(s_ref[1] * inv, Wn_ref[1]))
    out = jnp.maximum(out, 0.0)
    nrm = jnp.sqrt(jnp.sum(out * out, axis=1, keepdims=True))
    out = out / jnp.maximum(nrm, 1e-12)
    out_ref[0] = out[:, :FH]
    out_ref[1] = out[:, FH:]


def _tc_combine(sf, sums, deg2d, Wn):
    return pl.pallas_call(
        _combine_body,
        grid=(NPAD // _BM,),
        in_specs=[
            pl.BlockSpec((2, _BM, FH), lambda i: (0, i, 0)),
            pl.BlockSpec((2, _BM, FH), lambda i: (0, i, 0)),
            pl.BlockSpec((_BM, 1), lambda i: (i, 0)),
            pl.BlockSpec((2, FH, 2 * FH), lambda i: (0, 0, 0)),
        ],
        out_specs=pl.BlockSpec((2, _BM, FH), lambda i: (0, i, 0)),
        out_shape=jax.ShapeDtypeStruct((2, NPAD, FH), jnp.float32),
    )(sf, sums, deg2d, Wn)


def _head_body(h_ref,
               Wmv_ref, bmv_ref,
               Wd1_ref, bd1_ref, Wd2_ref, bd2_ref,
               Wr_ref, br_ref, Wsd_ref, bsd_ref,
               zloc_ref, zscale_ref, pxs_ref, pxr_ref, pxl_ref):
    L = zloc_ref.shape[1]
    C = pxs_ref.shape[1]
    h2 = jnp.concatenate([h_ref[0], h_ref[1]], axis=1)
    z = _dot(h2, Wmv_ref[...]) + bmv_ref[...]
    z_loc = z[:, :L]
    zloc_ref[...] = z_loc
    zscale_ref[...] = jnp.exp(z[:, L:]) + 1e-6
    px = jnp.maximum(_dot(z_loc, Wd1_ref[...]) + bd1_ref[...], 0.0)
    px = jnp.maximum(_dot(px, Wd2_ref[...]) + bd2_ref[...], 0.0)
    pxr_ref[...] = _dot(px, Wr_ref[...]) + br_ref[...]
    t = _dot(px, Wsd_ref[...]) + bsd_ref[...]
    ts = t[:, :C] - jnp.max(t[:, :C], axis=1, keepdims=True)
    e = jnp.exp(ts)
    pxs_ref[...] = e / jnp.sum(e, axis=1, keepdims=True)
    pxl_ref[...] = t[:, C:C + 1]


def _full_spec(shape):
    nd = len(shape)
    return pl.BlockSpec(shape, lambda i, _nd=nd: (0,) * _nd)


def _tc_head(h, W_mu, b_mu, W_var, b_var,
             Wd1, bd1, Wd2, bd2, W_scale, b_scale, W_r, b_r, W_do, b_do):
    L = W_mu.shape[1]
    C = W_scale.shape[1]
    F = W_r.shape[1]
    Wmv = jnp.concatenate([W_mu, W_var], axis=1)
    bmv = jnp.concatenate([b_mu, b_var]).reshape(1, 2 * L)
    Wsd = jnp.concatenate([W_scale, W_do], axis=1)
    bsd = jnp.concatenate([b_scale, b_do]).reshape(1, C + 1)
    return pl.pallas_call(
        _head_body,
        grid=(NPAD // _BM,),
        in_specs=[
            pl.BlockSpec((2, _BM, FH), lambda i: (0, i, 0)),
            _full_spec(Wmv.shape), _full_spec((1, 2 * L)),
            _full_spec(Wd1.shape), _full_spec((1, 2 * FH)),
            _full_spec(Wd2.shape), _full_spec((1, 2 * FH)),
            _full_spec(W_r.shape), _full_spec((1, F)),
            _full_spec(Wsd.shape), _full_spec((1, C + 1)),
        ],
        out_specs=[
            pl.BlockSpec((_BM, L), lambda i: (i, 0)),
            pl.BlockSpec((_BM, L), lambda i: (i, 0)),
            pl.BlockSpec((_BM, C), lambda i: (i, 0)),
            pl.BlockSpec((_BM, F), lambda i: (i, 0)),
            pl.BlockSpec((_BM, 1), lambda i: (i, 0)),
        ],
        out_shape=[
            jax.ShapeDtypeStruct((N, L), jnp.float32),
            jax.ShapeDtypeStruct((N, L), jnp.float32),
            jax.ShapeDtypeStruct((N, C), jnp.float32),
            jax.ShapeDtypeStruct((N, F), jnp.float32),
            jax.ShapeDtypeStruct((N, 1), jnp.float32),
        ],
    )(h, Wmv, bmv, Wd1, bd1.reshape(1, -1),
      Wd2, bd2.reshape(1, -1), W_r, b_r.reshape(1, F), Wsd, bsd)


def kernel(x, edge_index, Ws0, Wn0, b0, Ws1, Wn1, b1, W_mu, b_mu, W_var,
           b_var, Wd1, bd1, Wd2, bd2, W_scale, b_scale, W_r, b_r, W_do, b_do):
    src = edge_index[0]
    dst = edge_index[1]
    sc_aggregate = _get_sc_aggregate()

    h0 = _tc_pre(x)

    Wss = jnp.stack([Ws0.reshape(2, FH, 2 * FH), Ws1.reshape(2, FH, 2 * FH)])
    Wns = jnp.stack([Wn0.reshape(2, FH, 2 * FH), Wn1.reshape(2, FH, 2 * FH)])
    bs = jnp.stack([b0.reshape(1, 2 * FH), b1.reshape(1, 2 * FH)])

    def _scan_body(h, wk):
        Ws, Wn, b = wk
        sums, deg = sc_aggregate(h.reshape(2 * NPAD, FH), src, dst)
        sf = _tc_self(h, Ws, b)
        h_new = _tc_combine(sf, sums, deg.reshape(NPAD, 1), Wn)
        return h_new, None

    h2, _ = lax.scan(_scan_body, h0, (Wss, Wns, bs))
    return _tc_head(
        h2, W_mu, b_mu, W_var, b_var, Wd1, bd1, Wd2, bd2,
        W_scale, b_scale, W_r, b_r, W_do, b_do)


# speedup vs baseline: 7.0962x; 1.0077x over previous
import functools

import jax
import jax.numpy as jnp
from jax import lax
from jax.experimental import pallas as pl
from jax.experimental.pallas import tpu as pltpu
from jax.experimental.pallas import tpu_sc as plsc

N = 10000
NPAD = 10240
E = 160000
FH = 128
NSUB = 16
NCORE = 2
K = 128
EPT = E // NSUB
NCHUNK = EPT // K
TAIL = EPT - NCHUNK * K
NPR = NPAD // NSUB


@functools.cache
def _get_sc_aggregate():
    mesh = plsc.VectorSubcoreMesh(core_axis_name="c", subcore_axis_name="s",
                                  num_cores=NCORE, num_subcores=NSUB)

    @functools.partial(
        pl.kernel,
        out_type=[
            jax.ShapeDtypeStruct((NCORE, NPAD, FH), jnp.float32),
            jax.ShapeDtypeStruct((NPAD,), jnp.float32),
        ],
        mesh=mesh,
        scratch_types=[
            pltpu.VMEM((2, K), jnp.int32),
            pltpu.VMEM((2, K), jnp.int32),
            pltpu.VMEM((2, K, FH), jnp.float32),
            pltpu.VMEM((16,), jnp.int32),
            pltpu.VMEM((16,), jnp.int32),
            pltpu.VMEM((K,), jnp.float32),
            pltpu.VMEM((NPR,), jnp.float32),
            pltpu.VMEM_SHARED((NPAD, FH), jnp.float32),
            pltpu.VMEM_SHARED((NPAD,), jnp.float32),
        ] + [pltpu.SemaphoreType.DMA] * 8,
    )
    def _sc_aggregate(h_hbm, src_hbm, dst_hbm, sums_hbm, deg_hbm,
                      srcadj_v, dstb_v, rows_v, tsrc_v, tdst_v, ones_v,
                      zdeg_v, acc_sh, deg_sh, *sems):
        c = lax.axis_index("c")
        s = lax.axis_index("s")
        gsems = sems[0:2]
        ssems = sems[2:4]
        xsems = sems[4:6]
        dxsems = sems[6:8]

        def _zrow(i, carry):
            for j in range(FH // 16):
                rows_v[0, i, pl.ds(j * 16, 16)] = jnp.zeros((16,), jnp.float32)
            return carry
        lax.fori_loop(0, K, _zrow, 0)
        for j in range(K // 16):
            ones_v[pl.ds(j * 16, 16)] = jnp.ones((16,), jnp.float32)

        def _zdeg(i, carry):
            zdeg_v[pl.ds(i * 16, 16)] = jnp.zeros((16,), jnp.float32)
            return carry
        lax.fori_loop(0, NPR // 16, _zdeg, 0)

        for r in range(NPR // K):
            pltpu.async_copy(rows_v.at[0],
                             acc_sh.at[pl.ds(s * NPR + r * K, K), :],
                             gsems[0])
        for r in range(NPR // K):
            pltpu.make_async_copy(rows_v.at[0],
                                  acc_sh.at[pl.ds(s * NPR + r * K, K), :],
                                  gsems[0]).wait()
        pltpu.sync_copy(zdeg_v, deg_sh.at[pl.ds(s * NPR, NPR)])
        plsc.subcore_barrier()

        coff = c * NPAD

        def _adjust(b):
            for j in range(K // 16):
                srcadj_v[b, pl.ds(j * 16, 16)] = (
                    srcadj_v[b, pl.ds(j * 16, 16)] + coff)

        def _gather_start(b):
            pltpu.async_copy(h_hbm.at[srcadj_v.at[b]], rows_v.at[b], gsems[b])

        def _gather_wait(b):
            pltpu.make_async_copy(h_hbm.at[srcadj_v.at[b]], rows_v.at[b],
                                  gsems[b]).wait()

        def _srcidx_start(i, b):
            pltpu.async_copy(src_hbm.at[pl.ds(s * EPT + i * K, K)],
                             srcadj_v.at[b], xsems[b])

        def _srcidx_wait(i, b):
            pltpu.make_async_copy(src_hbm.at[pl.ds(s * EPT + i * K, K)],
                                  srcadj_v.at[b], xsems[b]).wait()

        def _dstidx_start(i, b):
            pltpu.async_copy(dst_hbm.at[pl.ds(s * EPT + i * K, K)],
                             dstb_v.at[b], dxsems[b])

        def _dstidx_wait(i, b):
            pltpu.make_async_copy(dst_hbm.at[pl.ds(s * EPT + i * K, K)],
                                  dstb_v.at[b], dxsems[b]).wait()

        for b in (0, 1):
            _srcidx_start(b, b)
            _dstidx_start(b, b)
            _srcidx_wait(b, b)
            _adjust(b)
            _gather_start(b)

        def _consume(i, b, nxt):
            _gather_wait(b)
            _dstidx_wait(i, b)
            sc = pltpu.async_copy(rows_v.at[b], acc_sh.at[dstb_v.at[b]],
                                  ssems[b], add=True)
            dg = pltpu.async_copy(ones_v, deg_sh.at[dstb_v.at[b]],
                                  ssems[b], add=True)
            if nxt:
                _srcidx_start(i + 2, b)
                _srcidx_wait(i + 2, b)
                _adjust(b)
            sc.wait()
            dg.wait()
            if nxt:
                _dstidx_start(i + 2, b)
                _gather_start(b)

        def _pair(p, carry):
            _consume(2 * p, 0, True)
            _consume(2 * p + 1, 1, True)
            return carry
        lax.fori_loop(0, NCHUNK // 2 - 1, _pair, 0)
        _consume(NCHUNK - 2, 0, False)
        _consume(NCHUNK - 1, 1, False)

        toff = s * EPT + NCHUNK * K
        pltpu.sync_copy(src_hbm.at[pl.ds(toff, TAIL)], tsrc_v)
        pltpu.sync_copy(dst_hbm.at[pl.ds(toff, TAIL)], tdst_v)
        tsrc_v[...] = tsrc_v[...] + coff
        pltpu.async_copy(h_hbm.at[tsrc_v], rows_v.at[0, pl.ds(0, TAIL)],
                         gsems[0]).wait()
        pltpu.sync_copy(rows_v.at[0, pl.ds(0, TAIL)], acc_sh.at[tdst_v],
                        add=True)
        pltpu.sync_copy(ones_v.at[pl.ds(0, TAIL)], deg_sh.at[tdst_v],
                        add=True)
        plsc.subcore_barrier()

        def _wb_out(r, b):
            return pltpu.make_async_copy(
                rows_v.at[b],
                sums_hbm.at[c, pl.ds(s * NPR + r * K, K), :], ssems[b])

        for r in range(NPR // K):
            b = r % 2
            if r >= 2:
                _wb_out(r - 2, b).wait()
            pltpu.async_copy(acc_sh.at[pl.ds(s * NPR + r * K, K), :],
                             rows_v.at[b], gsems[b]).wait()
            pltpu.async_copy(rows_v.at[b],
                             sums_hbm.at[c, pl.ds(s * NPR + r * K, K), :],
                             ssems[b])
        _wb_out(NPR // K - 2, (NPR // K - 2) % 2).wait()
        _wb_out(NPR // K - 1, (NPR // K - 1) % 2).wait()

        @pl.when(c == 0)
        def _():
            pltpu.sync_copy(deg_sh.at[pl.ds(s * NPR, NPR)], zdeg_v)
            pltpu.sync_copy(zdeg_v, deg_hbm.at[pl.ds(s * NPR, NPR)])

    return _sc_aggregate


_BM = 2560


def _pre_body(x_ref, out_ref):
    h = jnp.log(x_ref[...] + 1.0)
    out_ref[0] = h[:, :FH]
    out_ref[1] = h[:, FH:]


def _tc_pre(x):
    return pl.pallas_call(
        _pre_body,
        grid=(NPAD // _BM,),
        in_specs=[pl.BlockSpec((_BM, 2 * FH), lambda i: (i, 0))],
        out_specs=pl.BlockSpec((2, _BM, FH), lambda i: (0, i, 0)),
        out_shape=jax.ShapeDtypeStruct((2, NPAD, FH), jnp.float32),
    )(x)


def _dot(a, b):
    return jnp.dot(a, b, preferred_element_type=jnp.float32)


def _self_body(h_ref, Ws_ref, b_ref, out_ref):
    out = _dot(h_ref[0], Ws_ref[0]) + _dot(h_ref[1], Ws_ref[1]) + b_ref[...]
    out_ref[0] = out[:, :FH]
    out_ref[1] = out[:, FH:]


def _tc_self(h, Ws, b):
    return pl.pallas_call(
        _self_body,
        grid=(NPAD // _BM,),
        in_specs=[
            pl.BlockSpec((2, _BM, FH), lambda i: (0, i, 0)),
            pl.BlockSpec((2, FH, 2 * FH), lambda i: (0, 0, 0)),
            pl.BlockSpec((1, 2 * FH), lambda i: (0, 0)),
        ],
        out_specs=pl.BlockSpec((2, _BM, FH), lambda i: (0, i, 0)),
        out_shape=jax.ShapeDtypeStruct((2, NPAD, FH), jnp.float32),
    )(h, Ws, b)


def _combine_body(sf_ref, s_ref, deg_ref, Wn_ref, out_ref):
    inv = 1.0 / jnp.maximum(deg_ref[...], 1.0)
    out = (jnp.concatenate([sf_ref[0], sf_ref[1]], axis=1)
           + _dot(s_ref[0] * inv, Wn_ref[0]) + _dot(s_ref[1] * inv, Wn_ref[1]))
    out = jnp.maximum(out, 0.0)
    nrm = jnp.sqrt(jnp.sum(out * out, axis=1, keepdims=True))
    out = out / jnp.maximum(nrm, 1e-12)
    out_ref[0] = out[:, :FH]
    out_ref[1] = out[:, FH:]


def _tc_combine(sf, sums, deg2d, Wn):
    return pl.pallas_call(
        _combine_body,
        grid=(NPAD // _BM,),
        in_specs=[
            pl.BlockSpec((2, _BM, FH), lambda i: (0, i, 0)),
            pl.BlockSpec((2, _BM, FH), lambda i: (0, i, 0)),
            pl.BlockSpec((_BM, 1), lambda i: (i, 0)),
            pl.BlockSpec((2, FH, 2 * FH), lambda i: (0, 0, 0)),
        ],
        out_specs=pl.BlockSpec((2, _BM, FH), lambda i: (0, i, 0)),
        out_shape=jax.ShapeDtypeStruct((2, NPAD, FH), jnp.float32),
    )(sf, sums, deg2d, Wn)


def _head_body(h_ref,
               Wmv_ref, bmv_ref,
               Wd1_ref, bd1_ref, Wd2_ref, bd2_ref,
               Wr_ref, br_ref, Wsd_ref, bsd_ref,
               zloc_ref, zscale_ref, pxs_ref, pxr_ref, pxl_ref):
    L = zloc_ref.shape[1]
    C = pxs_ref.shape[1]
    h2 = jnp.concatenate([h_ref[0], h_ref[1]], axis=1)
    z = _dot(h2, Wmv_ref[...]) + bmv_ref[...]
    z_loc = z[:, :L]
    zloc_ref[...] = z_loc
    zscale_ref[...] = jnp.exp(z[:, L:]) + 1e-6
    px = jnp.maximum(_dot(z_loc, Wd1_ref[...]) + bd1_ref[...], 0.0)
    px = jnp.maximum(_dot(px, Wd2_ref[...]) + bd2_ref[...], 0.0)
    pxr_ref[...] = _dot(px, Wr_ref[...]) + br_ref[...]
    t = _dot(px, Wsd_ref[...]) + bsd_ref[...]
    ts = t[:, :C] - jnp.max(t[:, :C], axis=1, keepdims=True)
    e = jnp.exp(ts)
    pxs_ref[...] = e / jnp.sum(e, axis=1, keepdims=True)
    pxl_ref[...] = t[:, C:C + 1]


def _full_spec(shape):
    nd = len(shape)
    return pl.BlockSpec(shape, lambda i, _nd=nd: (0,) * _nd)


def _tc_head(h, W_mu, b_mu, W_var, b_var,
             Wd1, bd1, Wd2, bd2, W_scale, b_scale, W_r, b_r, W_do, b_do):
    L = W_mu.shape[1]
    C = W_scale.shape[1]
    F = W_r.shape[1]
    Wmv = jnp.concatenate([W_mu, W_var], axis=1)
    bmv = jnp.concatenate([b_mu, b_var]).reshape(1, 2 * L)
    Wsd = jnp.concatenate([W_scale, W_do], axis=1)
    bsd = jnp.concatenate([b_scale, b_do]).reshape(1, C + 1)
    return pl.pallas_call(
        _head_body,
        grid=(NPAD // _BM,),
        in_specs=[
            pl.BlockSpec((2, _BM, FH), lambda i: (0, i, 0)),
            _full_spec(Wmv.shape), _full_spec((1, 2 * L)),
            _full_spec(Wd1.shape), _full_spec((1, 2 * FH)),
            _full_spec(Wd2.shape), _full_spec((1, 2 * FH)),
            _full_spec(W_r.shape), _full_spec((1, F)),
            _full_spec(Wsd.shape), _full_spec((1, C + 1)),
        ],
        out_specs=[
            pl.BlockSpec((_BM, L), lambda i: (i, 0)),
            pl.BlockSpec((_BM, L), lambda i: (i, 0)),
            pl.BlockSpec((_BM, C), lambda i: (i, 0)),
            pl.BlockSpec((_BM, F), lambda i: (i, 0)),
            pl.BlockSpec((_BM, 1), lambda i: (i, 0)),
        ],
        out_shape=[
            jax.ShapeDtypeStruct((N, L), jnp.float32),
            jax.ShapeDtypeStruct((N, L), jnp.float32),
            jax.ShapeDtypeStruct((N, C), jnp.float32),
            jax.ShapeDtypeStruct((N, F), jnp.float32),
            jax.ShapeDtypeStruct((N, 1), jnp.float32),
        ],
    )(h, Wmv, bmv, Wd1, bd1.reshape(1, -1),
      Wd2, bd2.reshape(1, -1), W_r, b_r.reshape(1, F), Wsd, bsd)


def kernel(x, edge_index, Ws0, Wn0, b0, Ws1, Wn1, b1, W_mu, b_mu, W_var,
           b_var, Wd1, bd1, Wd2, bd2, W_scale, b_scale, W_r, b_r, W_do, b_do):
    src = edge_index[0]
    dst = edge_index[1]
    sc_aggregate = _get_sc_aggregate()

    h0 = _tc_pre(x)

    Wss = jnp.stack([Ws0.reshape(2, FH, 2 * FH), Ws1.reshape(2, FH, 2 * FH)])
    Wns = jnp.stack([Wn0.reshape(2, FH, 2 * FH), Wn1.reshape(2, FH, 2 * FH)])
    bs = jnp.stack([b0.reshape(1, 2 * FH), b1.reshape(1, 2 * FH)])

    def _scan_body(h, wk):
        Ws, Wn, b = wk
        sums, deg = sc_aggregate(h.reshape(2 * NPAD, FH), src, dst)
        sf = _tc_self(h, Ws, b)
        h_new = _tc_combine(sf, sums, deg.reshape(NPAD, 1), Wn)
        return h_new, None

    h2, _ = lax.scan(_scan_body, h0, (Wss, Wns, bs))
    return _tc_head(
        h2, W_mu, b_mu, W_var, b_var, Wd1, bd1, Wd2, bd2,
        W_scale, b_scale, W_r, b_r, W_do, b_do)
